# Initial kernel scaffold; baseline (speedup 1.0000x reference)
import jax
import jax.numpy as jnp
from jax.experimental import pallas as pl


def kernel(x, edge_index, W1, a_src1, a_dst1, b1, W2, a_src2, a_dst2, b2, W3, a_src3, a_dst3, b3):
    raise NotImplementedError("write your pallas kernel here")


# speedup vs baseline: 13.6372x; 13.6372x over previous
import functools

import jax
import jax.numpy as jnp
from jax import lax
from jax.experimental import pallas as pl
from jax.experimental.pallas import tpu as pltpu
from jax.experimental.pallas import tpu_sc as plsc

N = 10000
E = 320000
N_FEAT = 128
D_HID = 512
N_HEAD = 8
N_CLASS = 64

N_PAD = 10240
E_REAL = E + N
K = 128
E_PAD = 32 * K * 81
BN = 512
GRID = N_PAD // BN
NEG = -1e30

f32 = jnp.float32
i32 = jnp.int32


def _mask_pad_rows(asd, i):
    rows = i * BN + lax.broadcasted_iota(i32, (BN, 16), 0)
    return jnp.where(rows < N, asd, NEG)


def _accum_mm(mm_ref, asd, i):
    cur = jnp.max(asd, axis=0, keepdims=True)

    @pl.when(i == 0)
    def _():
        mm_ref[...] = cur

    @pl.when(i > 0)
    def _():
        mm_ref[...] = jnp.maximum(mm_ref[...], cur)


def _split_pairs(h):
    return jnp.stack([h[:, 0:128], h[:, 128:256], h[:, 256:384], h[:, 384:512]],
                     axis=0)


def _tc1_body(x_ref, w_ref, u_ref, hp_ref, asd_ref, mm_ref):
    i = pl.program_id(0)
    h = jnp.dot(x_ref[...], w_ref[...], preferred_element_type=f32)
    asd = jnp.dot(h, u_ref[...], preferred_element_type=f32)
    asd_ref[...] = _mask_pad_rows(asd, i)
    hp_ref[...] = _split_pairs(h)
    _accum_mm(mm_ref, asd_ref[...], i)


def _tc_mid_body(un_ref, den_ref, b_ref, res_ref, w_ref, u_ref, rsum_ref,
                 hp_ref, asd_ref, mm_ref, hact_ref, *, with_residual):
    i = pl.program_id(0)
    un = un_ref[...]
    hcat = jnp.concatenate([un[0], un[1], un[2], un[3]], axis=1)
    d = den_ref[0] + den_ref[1]
    dexp = jnp.dot(d, rsum_ref[...], preferred_element_type=f32)
    z = hcat / (dexp + 1e-16) + b_ref[...]
    hact = jnp.where(z > 0, z, jnp.exp(jnp.minimum(z, 0.0)) - 1.0)
    if with_residual:
        hact = hact + res_ref[...]
    hact_ref[...] = hact
    h2 = jnp.dot(hact, w_ref[...], preferred_element_type=f32)
    asd = jnp.dot(h2, u_ref[...], preferred_element_type=f32)
    asd_ref[...] = _mask_pad_rows(asd, i)
    hp_ref[...] = _split_pairs(h2) if h2.shape[1] == 512 else h2
    _accum_mm(mm_ref, asd_ref[...], i)


def _tc4_body(un_ref, den_ref, b_ref, out_ref):
    s = un_ref[0] + un_ref[1]
    d = den_ref[0][:, 0:1] + den_ref[1][:, 0:1]
    out_ref[...] = s / (d + 1e-16) + b_ref[...]


def _tc1(xp, W1, U1):
    return pl.pallas_call(
        _tc1_body,
        grid=(GRID,),
        in_specs=[
            pl.BlockSpec((BN, N_FEAT), lambda i: (i, 0)),
            pl.BlockSpec((N_FEAT, D_HID), lambda i: (0, 0)),
            pl.BlockSpec((D_HID, 16), lambda i: (0, 0)),
        ],
        out_specs=[
            pl.BlockSpec((4, BN, 128), lambda i: (0, i, 0)),
            pl.BlockSpec((BN, 16), lambda i: (i, 0)),
            pl.BlockSpec((1, 16), lambda i: (0, 0)),
        ],
        out_shape=[
            jax.ShapeDtypeStruct((4, N_PAD, 128), f32),
            jax.ShapeDtypeStruct((N_PAD, 16), f32),
            jax.ShapeDtypeStruct((1, 16), f32),
        ],
    )(xp, W1, U1)


def _tc_mid(un, den, b, res, W, U, RS, out_cols, with_residual):
    n_pairs = out_cols // 128 if out_cols >= 128 else 1
    hp_shape = (4, N_PAD, 128) if out_cols == 512 else (N_PAD, out_cols)
    hp_spec = (pl.BlockSpec((4, BN, 128), lambda i: (0, i, 0))
               if out_cols == 512 else pl.BlockSpec((BN, out_cols), lambda i: (i, 0)))
    body = functools.partial(_tc_mid_body, with_residual=with_residual)
    return pl.pallas_call(
        body,
        grid=(GRID,),
        in_specs=[
            pl.BlockSpec((4, BN, 128), lambda i: (0, i, 0)),
            pl.BlockSpec((2, BN, 8), lambda i: (0, i, 0)),
            pl.BlockSpec((1, D_HID), lambda i: (0, 0)),
            pl.BlockSpec((BN, D_HID), lambda i: (i, 0)),
            pl.BlockSpec((D_HID, out_cols), lambda i: (0, 0)),
            pl.BlockSpec((out_cols, 16), lambda i: (0, 0)),
            pl.BlockSpec((8, D_HID), lambda i: (0, 0)),
        ],
        out_specs=[
            hp_spec,
            pl.BlockSpec((BN, 16), lambda i: (i, 0)),
            pl.BlockSpec((1, 16), lambda i: (0, 0)),
            pl.BlockSpec((BN, D_HID), lambda i: (i, 0)),
        ],
        out_shape=[
            jax.ShapeDtypeStruct(hp_shape, f32),
            jax.ShapeDtypeStruct((N_PAD, 16), f32),
            jax.ShapeDtypeStruct((1, 16), f32),
            jax.ShapeDtypeStruct((N_PAD, D_HID), f32),
        ],
    )(un, den, b, res, W, U, RS)


def _tc4(un3, den3, b3):
    return pl.pallas_call(
        _tc4_body,
        grid=(GRID,),
        in_specs=[
            pl.BlockSpec((2, BN, 64), lambda i: (0, i, 0)),
            pl.BlockSpec((2, BN, 8), lambda i: (0, i, 0)),
            pl.BlockSpec((1, 64), lambda i: (0, 0)),
        ],
        out_specs=pl.BlockSpec((BN, 64), lambda i: (i, 0)),
        out_shape=jax.ShapeDtypeStruct((N_PAD, 64), f32),
    )(un3, den3, b3)


_MESH = dict(core_axis_name="c", subcore_axis_name="s", num_cores=2,
             num_subcores=16)

_SC_PARAMS = pltpu.CompilerParams(needs_layout_passes=False,
                                  use_tc_tiling_on_sc=False)

ROWS_PER_TILE = N_PAD // 16


def _leaky(v):
    return jnp.where(v < 0.0, 0.2 * v, v)


def _light_body(asd_hbm, mm_hbm, src_hbm, dst_hbm, zr_hbm,
                ex_hbm, den_hbm,
                idx_s, idx_d, asb, adb, exb, mmv, dacc, sem):
    c = lax.axis_index("c")
    s = lax.axis_index("s")
    lane = lax.iota(i32, 16)
    rem8 = lax.rem(lane, 8)
    row2 = lane // 8

    pltpu.sync_copy(mm_hbm, mmv)
    mt = plsc.load_gather(mmv, [rem8]) + plsc.load_gather(mmv, [rem8 + 8])
    m16 = _leaky(mt)

    pltpu.sync_copy(zr_hbm, dacc.at[pl.ds(ROWS_PER_TILE * s, ROWS_PER_TILE)])
    plsc.subcore_barrier()

    tile_base = c * (E_PAD // 2) + s * (E_PAD // 32)
    n_chunks = E_PAD // 32 // K

    def chunk(ci, carry):
        base = pl.multiple_of(tile_base + ci * K, K)
        pltpu.sync_copy(src_hbm.at[pl.ds(base, K)], idx_s)
        pltpu.sync_copy(dst_hbm.at[pl.ds(base, K)], idx_d)
        pltpu.async_copy(asd_hbm.at[idx_s], asb, sem).wait()
        pltpu.async_copy(asd_hbm.at[idx_d], adb, sem).wait()

        def pair(j, carry2):
            rowv = 2 * j + row2
            sval = plsc.load_gather(asb, [rowv, rem8])
            dval = plsc.load_gather(adb, [rowv, rem8 + 8])
            ex = jnp.exp(_leaky(sval + dval) - m16)
            plsc.store_scatter(exb, [rowv, rem8], ex)
            return carry2

        lax.fori_loop(0, K // 2, pair, 0, unroll=2)
        pltpu.sync_copy(exb, ex_hbm.at[pl.ds(base, K)])
        pltpu.sync_copy(exb, dacc.at[idx_d], add=True)
        return carry

    lax.fori_loop(0, n_chunks, chunk, 0)
    plsc.subcore_barrier()
    row0 = c * N_PAD + ROWS_PER_TILE * s
    pltpu.sync_copy(dacc.at[pl.ds(ROWS_PER_TILE * s, ROWS_PER_TILE)],
                    den_hbm.at[pl.ds(row0, ROWS_PER_TILE)])


def _make_light():
    return pl.kernel(
        _light_body,
        out_type=(
            jax.ShapeDtypeStruct((E_PAD, 8), f32),
            jax.ShapeDtypeStruct((2 * N_PAD, 8), f32),
        ),
        mesh=plsc.VectorSubcoreMesh(**_MESH),
        compiler_params=_SC_PARAMS,
        scratch_types=[
            pltpu.VMEM((K,), i32),
            pltpu.VMEM((K,), i32),
            pltpu.VMEM((K, 16), f32),
            pltpu.VMEM((K, 16), f32),
            pltpu.VMEM((K, 8), f32),
            pltpu.VMEM((16,), f32),
            pltpu.VMEM_SHARED((N_PAD, 8), f32),
            pltpu.SemaphoreType.DMA,
        ],
    )


def _heavy12_body(hp_hbm, ex_hbm, src_hbm, dst_hbm, zr_hbm,
                  out_hbm,
                  idx_s, adj, idx_d, exb, hh, msg, acc, sem):
    c = lax.axis_index("c")
    s = lax.axis_index("s")
    tile_base = s * (E_PAD // 16)
    n_chunks = E_PAD // 16 // K

    for r in (0, 1):
        P = 2 * c + r
        off = P * N_PAD
        pltpu.sync_copy(zr_hbm, acc.at[pl.ds(ROWS_PER_TILE * s, ROWS_PER_TILE)])
        plsc.subcore_barrier()

        def chunk(ci, carry):
            base = pl.multiple_of(tile_base + ci * K, K)
            pltpu.sync_copy(src_hbm.at[pl.ds(base, K)], idx_s)
            pltpu.sync_copy(dst_hbm.at[pl.ds(base, K)], idx_d)
            for j in range(K // 16):
                adj[pl.ds(16 * j, 16)] = idx_s[pl.ds(16 * j, 16)] + off
            pltpu.async_copy(hp_hbm.at[adj], hh, sem).wait()
            pltpu.sync_copy(ex_hbm.at[pl.ds(base, K)], exb)

            col0 = jnp.full((16,), 2 * P, dtype=i32)
            col1 = col0 + 1

            def edge(k, carry2):
                kv = jnp.full((16,), k, dtype=i32)
                a0 = plsc.load_gather(exb, [kv, col0])
                a1 = plsc.load_gather(exb, [kv, col1])
                for j in range(8):
                    sl = pl.ds(16 * j, 16)
                    msg[k, sl] = hh[k, sl] * (a0 if j < 4 else a1)
                return carry2

            lax.fori_loop(0, K, edge, 0, unroll=2)
            pltpu.sync_copy(msg, acc.at[idx_d], add=True)
            return carry

        lax.fori_loop(0, n_chunks, chunk, 0)
        plsc.subcore_barrier()
        pltpu.sync_copy(acc.at[pl.ds(ROWS_PER_TILE * s, ROWS_PER_TILE)],
                        out_hbm.at[pl.ds(off + ROWS_PER_TILE * s, ROWS_PER_TILE)])
        plsc.subcore_barrier()


def _make_heavy12():
    return pl.kernel(
        _heavy12_body,
        out_type=jax.ShapeDtypeStruct((4 * N_PAD, 128), f32),
        mesh=plsc.VectorSubcoreMesh(**_MESH),
        compiler_params=_SC_PARAMS,
        scratch_types=[
            pltpu.VMEM((K,), i32),
            pltpu.VMEM((K,), i32),
            pltpu.VMEM((K,), i32),
            pltpu.VMEM((K, 8), f32),
            pltpu.VMEM((K, 128), f32),
            pltpu.VMEM((K, 128), f32),
            pltpu.VMEM_SHARED((N_PAD, 128), f32),
            pltpu.SemaphoreType.DMA,
        ],
    )


def _heavy3_body(h3_hbm, ex_hbm, src_hbm, dst_hbm, zr_hbm,
                 out_hbm,
                 idx_s, idx_d, exb, hh, msg, acc, sem):
    c = lax.axis_index("c")
    s = lax.axis_index("s")
    pltpu.sync_copy(zr_hbm, acc.at[pl.ds(ROWS_PER_TILE * s, ROWS_PER_TILE)])
    plsc.subcore_barrier()

    tile_base = c * (E_PAD // 2) + s * (E_PAD // 32)
    n_chunks = E_PAD // 32 // K

    def chunk(ci, carry):
        base = pl.multiple_of(tile_base + ci * K, K)
        pltpu.sync_copy(src_hbm.at[pl.ds(base, K)], idx_s)
        pltpu.sync_copy(dst_hbm.at[pl.ds(base, K)], idx_d)
        pltpu.async_copy(h3_hbm.at[idx_s], hh, sem).wait()
        pltpu.sync_copy(ex_hbm.at[pl.ds(base, K)], exb)

        col0 = jnp.full((16,), 0, dtype=i32)

        def edge(k, carry2):
            kv = jnp.full((16,), k, dtype=i32)
            a0 = plsc.load_gather(exb, [kv, col0])
            for j in range(4):
                sl = pl.ds(16 * j, 16)
                msg[k, sl] = hh[k, sl] * a0
            return carry2

        lax.fori_loop(0, K, edge, 0, unroll=2)
        pltpu.sync_copy(msg, acc.at[idx_d], add=True)
        return carry

    lax.fori_loop(0, n_chunks, chunk, 0)
    plsc.subcore_barrier()
    row0 = c * N_PAD + ROWS_PER_TILE * s
    pltpu.sync_copy(acc.at[pl.ds(ROWS_PER_TILE * s, ROWS_PER_TILE)],
                    out_hbm.at[pl.ds(row0, ROWS_PER_TILE)])


def _make_heavy3():
    return pl.kernel(
        _heavy3_body,
        out_type=jax.ShapeDtypeStruct((2 * N_PAD, 64), f32),
        mesh=plsc.VectorSubcoreMesh(**_MESH),
        compiler_params=_SC_PARAMS,
        scratch_types=[
            pltpu.VMEM((K,), i32),
            pltpu.VMEM((K,), i32),
            pltpu.VMEM((K, 8), f32),
            pltpu.VMEM((K, 64), f32),
            pltpu.VMEM((K, 64), f32),
            pltpu.VMEM_SHARED((N_PAD, 64), f32),
            pltpu.SemaphoreType.DMA,
        ],
    )


def _proj_matrix(a_src, a_dst):
    heads, ch = a_src.shape
    d = heads * ch
    sel = (jnp.arange(d)[:, None] // ch == jnp.arange(heads)[None, :])
    us = jnp.where(sel, a_src.reshape(d)[:, None], 0.0)
    ud = jnp.where(sel, a_dst.reshape(d)[:, None], 0.0)
    z = jnp.zeros((d, 8 - heads), dtype=f32)
    return jnp.concatenate([us, z, ud, z], axis=1).astype(f32)


@jax.jit
def kernel(x, edge_index, W1, a_src1, a_dst1, b1, W2, a_src2, a_dst2, b2,
           W3, a_src3, a_dst3, b3):
    xp = jnp.zeros((N_PAD, N_FEAT), f32).at[:N].set(x)
    loop = jnp.arange(N, dtype=i32)
    padi = jnp.full((E_PAD - E_REAL,), N, dtype=i32)
    src = jnp.concatenate([edge_index[0].astype(i32), loop, padi])
    dst = jnp.concatenate([edge_index[1].astype(i32), loop, padi])

    U1 = _proj_matrix(a_src1, a_dst1)
    U2 = _proj_matrix(a_src2, a_dst2)
    U3 = _proj_matrix(a_src3, a_dst3)
    RS = (jnp.arange(D_HID)[None, :] // 64 == jnp.arange(8)[:, None]).astype(f32)

    z8 = jnp.zeros((ROWS_PER_TILE, 8), f32)
    z64 = jnp.zeros((ROWS_PER_TILE, 64), f32)
    z128 = jnp.zeros((ROWS_PER_TILE, 128), f32)

    light = _make_light()
    heavy12 = _make_heavy12()
    heavy3 = _make_heavy3()

    hp1, asd1, mm1 = _tc1(xp, W1, U1)
    ex1, den1 = light(asd1, mm1.reshape(16), src, dst, z8)
    un1 = heavy12(hp1.reshape(4 * N_PAD, 128), ex1, src, dst, z128)

    dummy_res = jnp.zeros((N_PAD, D_HID), f32)
    hp2, asd2, mm2, h1a = _tc_mid(un1.reshape(4, N_PAD, 128),
                                  den1.reshape(2, N_PAD, 8),
                                  b1.reshape(1, D_HID), dummy_res,
                                  W2, U2, RS, 512, with_residual=False)
    ex2, den2 = light(asd2, mm2.reshape(16), src, dst, z8)
    un2 = heavy12(hp2.reshape(4 * N_PAD, 128), ex2, src, dst, z128)

    h3t, asd3, mm3, _ = _tc_mid(un2.reshape(4, N_PAD, 128),
                                den2.reshape(2, N_PAD, 8),
                                b2.reshape(1, D_HID), h1a,
                                W3, U3, RS, 64, with_residual=True)
    ex3, den3 = light(asd3, mm3.reshape(16), src, dst, z8)
    un3 = heavy3(h3t, ex3, src, dst, z64)

    out = _tc4(un3.reshape(2, N_PAD, 64), den3.reshape(2, N_PAD, 8),
               b3.reshape(1, 64))
    return out[:N]


# speedup vs baseline: 27.7177x; 2.0325x over previous
import functools

import jax
import jax.numpy as jnp
from jax import lax
from jax.experimental import pallas as pl
from jax.experimental.pallas import tpu as pltpu
from jax.experimental.pallas import tpu_sc as plsc

N = 10000
E = 320000
N_FEAT = 128
D_HID = 512
N_HEAD = 8
N_CLASS = 64

N_PAD = 10240
E_REAL = E + N
K = 128
E_PAD = 32 * K * 81
BN = 512
GRID = N_PAD // BN
NEG = -1e30

f32 = jnp.float32
i32 = jnp.int32


def _mask_pad_rows(asd, i):
    rows = i * BN + lax.broadcasted_iota(i32, (BN, 16), 0)
    return jnp.where(rows < N, asd, NEG)


def _accum_mm(mm_ref, asd, i):
    cur = jnp.max(asd, axis=0, keepdims=True)

    @pl.when(i == 0)
    def _():
        mm_ref[...] = cur

    @pl.when(i > 0)
    def _():
        mm_ref[...] = jnp.maximum(mm_ref[...], cur)


def _split_pairs(h):
    return jnp.stack([h[:, 0:128], h[:, 128:256], h[:, 256:384], h[:, 384:512]],
                     axis=0)


def _tc1_body(x_ref, w_ref, u_ref, hp_ref, asd_ref, mm_ref):
    i = pl.program_id(0)
    h = jnp.dot(x_ref[...], w_ref[...], preferred_element_type=f32)
    asd = jnp.dot(h, u_ref[...], preferred_element_type=f32)
    asd_ref[...] = _mask_pad_rows(asd, i)
    hp_ref[...] = _split_pairs(h)
    _accum_mm(mm_ref, asd_ref[...], i)


def _tc_mid_body(un_ref, den_ref, b_ref, res_ref, w_ref, u_ref, rsum_ref,
                 hp_ref, asd_ref, mm_ref, hact_ref, *, with_residual):
    i = pl.program_id(0)
    un = un_ref[...]
    hcat = jnp.concatenate([un[0], un[1], un[2], un[3]], axis=1)
    d = den_ref[0] + den_ref[1]
    dexp = jnp.dot(d, rsum_ref[...], preferred_element_type=f32)
    z = hcat / (dexp + 1e-16) + b_ref[...]
    hact = jnp.where(z > 0, z, jnp.exp(jnp.minimum(z, 0.0)) - 1.0)
    if with_residual:
        hact = hact + res_ref[...]
    hact_ref[...] = hact
    h2 = jnp.dot(hact, w_ref[...], preferred_element_type=f32)
    asd = jnp.dot(h2, u_ref[...], preferred_element_type=f32)
    asd_ref[...] = _mask_pad_rows(asd, i)
    hp_ref[...] = _split_pairs(h2) if h2.shape[1] == 512 else h2
    _accum_mm(mm_ref, asd_ref[...], i)


def _tc4_body(un_ref, den_ref, b_ref, out_ref):
    s = un_ref[0] + un_ref[1]
    d = den_ref[0][:, 0:1] + den_ref[1][:, 0:1]
    out_ref[...] = s / (d + 1e-16) + b_ref[...]


def _tc1(xp, W1, U1):
    return pl.pallas_call(
        _tc1_body,
        grid=(GRID,),
        in_specs=[
            pl.BlockSpec((BN, N_FEAT), lambda i: (i, 0)),
            pl.BlockSpec((N_FEAT, D_HID), lambda i: (0, 0)),
            pl.BlockSpec((D_HID, 16), lambda i: (0, 0)),
        ],
        out_specs=[
            pl.BlockSpec((4, BN, 128), lambda i: (0, i, 0)),
            pl.BlockSpec((BN, 16), lambda i: (i, 0)),
            pl.BlockSpec((1, 16), lambda i: (0, 0)),
        ],
        out_shape=[
            jax.ShapeDtypeStruct((4, N_PAD, 128), f32),
            jax.ShapeDtypeStruct((N_PAD, 16), f32),
            jax.ShapeDtypeStruct((1, 16), f32),
        ],
    )(xp, W1, U1)


def _tc_mid(un, den, b, res, W, U, RS, out_cols, with_residual):
    n_pairs = out_cols // 128 if out_cols >= 128 else 1
    hp_shape = (4, N_PAD, 128) if out_cols == 512 else (N_PAD, out_cols)
    hp_spec = (pl.BlockSpec((4, BN, 128), lambda i: (0, i, 0))
               if out_cols == 512 else pl.BlockSpec((BN, out_cols), lambda i: (i, 0)))
    body = functools.partial(_tc_mid_body, with_residual=with_residual)
    return pl.pallas_call(
        body,
        grid=(GRID,),
        in_specs=[
            pl.BlockSpec((4, BN, 128), lambda i: (0, i, 0)),
            pl.BlockSpec((2, BN, 8), lambda i: (0, i, 0)),
            pl.BlockSpec((1, D_HID), lambda i: (0, 0)),
            pl.BlockSpec((BN, D_HID), lambda i: (i, 0)),
            pl.BlockSpec((D_HID, out_cols), lambda i: (0, 0)),
            pl.BlockSpec((out_cols, 16), lambda i: (0, 0)),
            pl.BlockSpec((8, D_HID), lambda i: (0, 0)),
        ],
        out_specs=[
            hp_spec,
            pl.BlockSpec((BN, 16), lambda i: (i, 0)),
            pl.BlockSpec((1, 16), lambda i: (0, 0)),
            pl.BlockSpec((BN, D_HID), lambda i: (i, 0)),
        ],
        out_shape=[
            jax.ShapeDtypeStruct(hp_shape, f32),
            jax.ShapeDtypeStruct((N_PAD, 16), f32),
            jax.ShapeDtypeStruct((1, 16), f32),
            jax.ShapeDtypeStruct((N_PAD, D_HID), f32),
        ],
    )(un, den, b, res, W, U, RS)


def _tc4(un3, den3, b3):
    return pl.pallas_call(
        _tc4_body,
        grid=(GRID,),
        in_specs=[
            pl.BlockSpec((2, BN, 64), lambda i: (0, i, 0)),
            pl.BlockSpec((2, BN, 8), lambda i: (0, i, 0)),
            pl.BlockSpec((1, 64), lambda i: (0, 0)),
        ],
        out_specs=pl.BlockSpec((BN, 64), lambda i: (i, 0)),
        out_shape=jax.ShapeDtypeStruct((N_PAD, 64), f32),
    )(un3, den3, b3)


_MESH = dict(core_axis_name="c", subcore_axis_name="s", num_cores=2,
             num_subcores=16)

_SC_PARAMS = pltpu.CompilerParams(needs_layout_passes=False,
                                  use_tc_tiling_on_sc=False)

ROWS_PER_TILE = N_PAD // 16


def _leaky(v):
    return jnp.where(v < 0.0, 0.2 * v, v)


def _light_body(asd_hbm, mm_hbm, src_hbm, dst_hbm, zr_hbm,
                ex_hbm, den_hbm,
                idx_s, idx_d, asb, adb, exb, mmv, dacc, sem):
    c = lax.axis_index("c")
    s = lax.axis_index("s")
    lane = lax.iota(i32, 16)
    rem8 = lax.rem(lane, 8)
    row2 = lane // 8

    pltpu.sync_copy(mm_hbm, mmv)
    mt = plsc.load_gather(mmv, [rem8]) + plsc.load_gather(mmv, [rem8 + 8])
    m16 = _leaky(mt)

    pltpu.sync_copy(zr_hbm, dacc.at[pl.ds(ROWS_PER_TILE * s, ROWS_PER_TILE)])
    plsc.subcore_barrier()

    tile_base = c * (E_PAD // 2) + s * (E_PAD // 32)
    n_chunks = E_PAD // 32 // K

    def chunk(ci, carry):
        base = pl.multiple_of(tile_base + ci * K, K)
        pltpu.sync_copy(src_hbm.at[pl.ds(base, K)], idx_s)
        pltpu.sync_copy(dst_hbm.at[pl.ds(base, K)], idx_d)
        pltpu.async_copy(asd_hbm.at[idx_s], asb, sem).wait()
        pltpu.async_copy(asd_hbm.at[idx_d], adb, sem).wait()

        def pair(j, carry2):
            rowv = 2 * j + row2
            sval = plsc.load_gather(asb, [rowv, rem8])
            dval = plsc.load_gather(adb, [rowv, rem8 + 8])
            ex = jnp.exp(_leaky(sval + dval) - m16)
            plsc.store_scatter(exb, [rowv, rem8], ex)
            return carry2

        lax.fori_loop(0, K // 2, pair, 0, unroll=2)
        pltpu.sync_copy(exb, ex_hbm.at[pl.ds(base, K)])
        pltpu.sync_copy(exb, dacc.at[idx_d], add=True)
        return carry

    lax.fori_loop(0, n_chunks, chunk, 0)
    plsc.subcore_barrier()
    row0 = c * N_PAD + ROWS_PER_TILE * s
    pltpu.sync_copy(dacc.at[pl.ds(ROWS_PER_TILE * s, ROWS_PER_TILE)],
                    den_hbm.at[pl.ds(row0, ROWS_PER_TILE)])


def _make_light():
    return pl.kernel(
        _light_body,
        out_type=(
            jax.ShapeDtypeStruct((E_PAD, 8), f32),
            jax.ShapeDtypeStruct((2 * N_PAD, 8), f32),
        ),
        mesh=plsc.VectorSubcoreMesh(**_MESH),
        compiler_params=_SC_PARAMS,
        scratch_types=[
            pltpu.VMEM((K,), i32),
            pltpu.VMEM((K,), i32),
            pltpu.VMEM((K, 16), f32),
            pltpu.VMEM((K, 16), f32),
            pltpu.VMEM((K, 8), f32),
            pltpu.VMEM((16,), f32),
            pltpu.VMEM_SHARED((N_PAD, 8), f32),
            pltpu.SemaphoreType.DMA,
        ],
    )


def _heavy12_body(hp_hbm, ex_hbm, src_hbm, dst_hbm, zr_hbm,
                  out_hbm,
                  idx_s2, adj2, idx_d2, exb2, hh2, acc,
                  semi0, semi1, semg0, semg1):
    c = lax.axis_index("c")
    s = lax.axis_index("s")
    tile_base = s * (E_PAD // 16)
    n_chunks = E_PAD // 16 // K
    max_base = E_PAD - K
    semi = (semi0, semi1)
    semg = (semg0, semg1)

    def cbase(ci):
        return pl.multiple_of(jnp.minimum(tile_base + ci * K, max_base), K)

    def start_idx(ci, b):
        base = cbase(ci)
        pltpu.async_copy(src_hbm.at[pl.ds(base, K)], idx_s2.at[b], semi[b])
        pltpu.async_copy(dst_hbm.at[pl.ds(base, K)], idx_d2.at[b], semi[b])
        pltpu.async_copy(ex_hbm.at[pl.ds(base, K)], exb2.at[b], semi[b])

    def wait_idx(b):
        pltpu.make_async_copy(src_hbm.at[pl.ds(0, K)], idx_s2.at[b], semi[b]).wait()
        pltpu.make_async_copy(dst_hbm.at[pl.ds(0, K)], idx_d2.at[b], semi[b]).wait()
        pltpu.make_async_copy(ex_hbm.at[pl.ds(0, K)], exb2.at[b], semi[b]).wait()

    def start_gather(off, b):
        for j in range(K // 16):
            sl = pl.ds(16 * j, 16)
            adj2[b, sl] = jnp.minimum(idx_s2[b, sl], N_PAD - 1) + off
        pltpu.async_copy(hp_hbm.at[adj2.at[b]], hh2.at[b], semg[b])

    def wait_gather(b):
        pltpu.make_async_copy(hp_hbm.at[adj2.at[b]], hh2.at[b], semg[b]).wait()

    for r in (0, 1):
        P = 2 * c + r
        off = P * N_PAD
        pltpu.sync_copy(zr_hbm, acc.at[pl.ds(ROWS_PER_TILE * s, ROWS_PER_TILE)])
        plsc.subcore_barrier()

        col0 = jnp.full((16,), 2 * P, dtype=i32)
        col1 = col0 + 1

        start_idx(0, 0)
        wait_idx(0)
        start_gather(off, 0)
        start_idx(1, 1)

        def chunk(g, carry):
            for b in (0, 1):
                nb = 1 - b
                wait_idx(nb)
                start_gather(off, nb)
                wait_gather(b)

                def edge(k, carry2):
                    kv = jnp.full((16,), k, dtype=i32)
                    a0 = plsc.load_gather(exb2.at[b], [kv, col0])
                    a1 = plsc.load_gather(exb2.at[b], [kv, col1])
                    for j in range(8):
                        sl = pl.ds(16 * j, 16)
                        hh2[b, k, sl] = hh2[b, k, sl] * (a0 if j < 4 else a1)
                    return carry2

                lax.fori_loop(0, K, edge, 0, unroll=2)
                pltpu.sync_copy(hh2.at[b], acc.at[idx_d2.at[b]], add=True)
                start_idx(2 * g + b + 2, b)
            return carry

        lax.fori_loop(0, n_chunks // 2, chunk, 0)
        wait_gather(0)
        wait_idx(1)
        plsc.subcore_barrier()
        pltpu.sync_copy(acc.at[pl.ds(ROWS_PER_TILE * s, ROWS_PER_TILE)],
                        out_hbm.at[pl.ds(off + ROWS_PER_TILE * s, ROWS_PER_TILE)])
        plsc.subcore_barrier()


def _make_heavy12():
    return pl.kernel(
        _heavy12_body,
        out_type=jax.ShapeDtypeStruct((4 * N_PAD, 128), f32),
        mesh=plsc.VectorSubcoreMesh(**_MESH),
        compiler_params=_SC_PARAMS,
        scratch_types=[
            pltpu.VMEM((2, K), i32),
            pltpu.VMEM((2, K), i32),
            pltpu.VMEM((2, K), i32),
            pltpu.VMEM((2, K, 8), f32),
            pltpu.VMEM((2, K, 128), f32),
            pltpu.VMEM_SHARED((N_PAD, 128), f32),
            pltpu.SemaphoreType.DMA,
            pltpu.SemaphoreType.DMA,
            pltpu.SemaphoreType.DMA,
            pltpu.SemaphoreType.DMA,
        ],
    )


def _heavy3_body(h3_hbm, ex_hbm, src_hbm, dst_hbm, zr_hbm,
                 out_hbm,
                 idx_s, idx_d, exb, hh, msg, acc, sem):
    c = lax.axis_index("c")
    s = lax.axis_index("s")
    pltpu.sync_copy(zr_hbm, acc.at[pl.ds(ROWS_PER_TILE * s, ROWS_PER_TILE)])
    plsc.subcore_barrier()

    tile_base = c * (E_PAD // 2) + s * (E_PAD // 32)
    n_chunks = E_PAD // 32 // K

    def chunk(ci, carry):
        base = pl.multiple_of(tile_base + ci * K, K)
        pltpu.sync_copy(src_hbm.at[pl.ds(base, K)], idx_s)
        pltpu.sync_copy(dst_hbm.at[pl.ds(base, K)], idx_d)
        pltpu.async_copy(h3_hbm.at[idx_s], hh, sem).wait()
        pltpu.sync_copy(ex_hbm.at[pl.ds(base, K)], exb)

        col0 = jnp.full((16,), 0, dtype=i32)

        def edge(k, carry2):
            kv = jnp.full((16,), k, dtype=i32)
            a0 = plsc.load_gather(exb, [kv, col0])
            for j in range(4):
                sl = pl.ds(16 * j, 16)
                msg[k, sl] = hh[k, sl] * a0
            return carry2

        lax.fori_loop(0, K, edge, 0, unroll=2)
        pltpu.sync_copy(msg, acc.at[idx_d], add=True)
        return carry

    lax.fori_loop(0, n_chunks, chunk, 0)
    plsc.subcore_barrier()
    row0 = c * N_PAD + ROWS_PER_TILE * s
    pltpu.sync_copy(acc.at[pl.ds(ROWS_PER_TILE * s, ROWS_PER_TILE)],
                    out_hbm.at[pl.ds(row0, ROWS_PER_TILE)])


def _make_heavy3():
    return pl.kernel(
        _heavy3_body,
        out_type=jax.ShapeDtypeStruct((2 * N_PAD, 64), f32),
        mesh=plsc.VectorSubcoreMesh(**_MESH),
        compiler_params=_SC_PARAMS,
        scratch_types=[
            pltpu.VMEM((K,), i32),
            pltpu.VMEM((K,), i32),
            pltpu.VMEM((K, 8), f32),
            pltpu.VMEM((K, 64), f32),
            pltpu.VMEM((K, 64), f32),
            pltpu.VMEM_SHARED((N_PAD, 64), f32),
            pltpu.SemaphoreType.DMA,
        ],
    )


def _proj_matrix(a_src, a_dst):
    heads, ch = a_src.shape
    d = heads * ch
    sel = (jnp.arange(d)[:, None] // ch == jnp.arange(heads)[None, :])
    us = jnp.where(sel, a_src.reshape(d)[:, None], 0.0)
    ud = jnp.where(sel, a_dst.reshape(d)[:, None], 0.0)
    z = jnp.zeros((d, 8 - heads), dtype=f32)
    return jnp.concatenate([us, z, ud, z], axis=1).astype(f32)


@jax.jit
def kernel(x, edge_index, W1, a_src1, a_dst1, b1, W2, a_src2, a_dst2, b2,
           W3, a_src3, a_dst3, b3):
    xp = jnp.zeros((N_PAD, N_FEAT), f32).at[:N].set(x)
    loop = jnp.arange(N, dtype=i32)
    padi = jnp.full((E_PAD - E_REAL,), N, dtype=i32)
    src = jnp.concatenate([edge_index[0].astype(i32), loop, padi])
    dst = jnp.concatenate([edge_index[1].astype(i32), loop, padi])

    U1 = _proj_matrix(a_src1, a_dst1)
    U2 = _proj_matrix(a_src2, a_dst2)
    U3 = _proj_matrix(a_src3, a_dst3)
    RS = (jnp.arange(D_HID)[None, :] // 64 == jnp.arange(8)[:, None]).astype(f32)

    z8 = jnp.zeros((ROWS_PER_TILE, 8), f32)
    z64 = jnp.zeros((ROWS_PER_TILE, 64), f32)
    z128 = jnp.zeros((ROWS_PER_TILE, 128), f32)

    light = _make_light()
    heavy12 = _make_heavy12()
    heavy3 = _make_heavy3()

    hp1, asd1, mm1 = _tc1(xp, W1, U1)
    ex1, den1 = light(asd1, mm1.reshape(16), src, dst, z8)
    un1 = heavy12(hp1.reshape(4 * N_PAD, 128), ex1, src, dst, z128)

    dummy_res = jnp.zeros((N_PAD, D_HID), f32)
    hp2, asd2, mm2, h1a = _tc_mid(un1.reshape(4, N_PAD, 128),
                                  den1.reshape(2, N_PAD, 8),
                                  b1.reshape(1, D_HID), dummy_res,
                                  W2, U2, RS, 512, with_residual=False)
    ex2, den2 = light(asd2, mm2.reshape(16), src, dst, z8)
    un2 = heavy12(hp2.reshape(4 * N_PAD, 128), ex2, src, dst, z128)

    h3t, asd3, mm3, _ = _tc_mid(un2.reshape(4, N_PAD, 128),
                                den2.reshape(2, N_PAD, 8),
                                b2.reshape(1, D_HID), h1a,
                                W3, U3, RS, 64, with_residual=True)
    ex3, den3 = light(asd3, mm3.reshape(16), src, dst, z8)
    un3 = heavy3(h3t, ex3, src, dst, z64)

    out = _tc4(un3.reshape(2, N_PAD, 64), den3.reshape(2, N_PAD, 8),
               b3.reshape(1, 64))
    return out[:N]


# speedup vs baseline: 33.2903x; 1.2011x over previous
import functools

import jax
import jax.numpy as jnp
from jax import lax
from jax.experimental import pallas as pl
from jax.experimental.pallas import tpu as pltpu
from jax.experimental.pallas import tpu_sc as plsc

N = 10000
E = 320000
N_FEAT = 128
D_HID = 512
N_HEAD = 8
N_CLASS = 64

N_PAD = 10240
E_REAL = E + N
K = 128
E_PAD = 32 * K * 81
BN = 512
GRID = N_PAD // BN
NEG = -1e30

f32 = jnp.float32
i32 = jnp.int32


def _mask_pad_rows(asd, i):
    rows = i * BN + lax.broadcasted_iota(i32, (BN, 16), 0)
    return jnp.where(rows < N, asd, NEG)


def _accum_mm(mm_ref, asd, i):
    cur = jnp.max(asd, axis=0, keepdims=True)

    @pl.when(i == 0)
    def _():
        mm_ref[...] = cur

    @pl.when(i > 0)
    def _():
        mm_ref[...] = jnp.maximum(mm_ref[...], cur)


def _split_pairs(h):
    return jnp.stack([h[:, 0:128], h[:, 128:256], h[:, 256:384], h[:, 384:512]],
                     axis=0)


def _tc1_body(x_ref, w_ref, u_ref, hp_ref, asd_ref, mm_ref):
    i = pl.program_id(0)
    h = jnp.dot(x_ref[...], w_ref[...], preferred_element_type=f32)
    asd = jnp.dot(h, u_ref[...], preferred_element_type=f32)
    asd_ref[...] = _mask_pad_rows(asd, i)
    hp_ref[...] = _split_pairs(h)
    _accum_mm(mm_ref, asd_ref[...], i)


def _tc_mid_body(un_ref, den_ref, b_ref, res_ref, w_ref, u_ref, rsum_ref,
                 hp_ref, asd_ref, mm_ref, hact_ref, *, with_residual):
    i = pl.program_id(0)
    un = un_ref[...]
    hcat = jnp.concatenate([un[0], un[1], un[2], un[3]], axis=1)
    d = den_ref[0] + den_ref[1]
    dexp = jnp.dot(d, rsum_ref[...], preferred_element_type=f32)
    z = hcat / (dexp + 1e-16) + b_ref[...]
    hact = jnp.where(z > 0, z, jnp.exp(jnp.minimum(z, 0.0)) - 1.0)
    if with_residual:
        hact = hact + res_ref[...]
    hact_ref[...] = hact
    h2 = jnp.dot(hact, w_ref[...], preferred_element_type=f32)
    asd = jnp.dot(h2, u_ref[...], preferred_element_type=f32)
    asd_ref[...] = _mask_pad_rows(asd, i)
    hp_ref[...] = _split_pairs(h2) if h2.shape[1] == 512 else h2
    _accum_mm(mm_ref, asd_ref[...], i)


def _tc4_body(un_ref, den_ref, b_ref, out_ref):
    s = un_ref[0] + un_ref[1]
    d = den_ref[0][:, 0:1] + den_ref[1][:, 0:1]
    out_ref[...] = s / (d + 1e-16) + b_ref[...]


def _tc1(xp, W1, U1):
    return pl.pallas_call(
        _tc1_body,
        grid=(GRID,),
        in_specs=[
            pl.BlockSpec((BN, N_FEAT), lambda i: (i, 0)),
            pl.BlockSpec((N_FEAT, D_HID), lambda i: (0, 0)),
            pl.BlockSpec((D_HID, 16), lambda i: (0, 0)),
        ],
        out_specs=[
            pl.BlockSpec((4, BN, 128), lambda i: (0, i, 0)),
            pl.BlockSpec((BN, 16), lambda i: (i, 0)),
            pl.BlockSpec((1, 16), lambda i: (0, 0)),
        ],
        out_shape=[
            jax.ShapeDtypeStruct((4, N_PAD, 128), f32),
            jax.ShapeDtypeStruct((N_PAD, 16), f32),
            jax.ShapeDtypeStruct((1, 16), f32),
        ],
    )(xp, W1, U1)


def _tc_mid(un, den, b, res, W, U, RS, out_cols, with_residual):
    n_pairs = out_cols // 128 if out_cols >= 128 else 1
    hp_shape = (4, N_PAD, 128) if out_cols == 512 else (N_PAD, out_cols)
    hp_spec = (pl.BlockSpec((4, BN, 128), lambda i: (0, i, 0))
               if out_cols == 512 else pl.BlockSpec((BN, out_cols), lambda i: (i, 0)))
    body = functools.partial(_tc_mid_body, with_residual=with_residual)
    return pl.pallas_call(
        body,
        grid=(GRID,),
        in_specs=[
            pl.BlockSpec((4, BN, 128), lambda i: (0, i, 0)),
            pl.BlockSpec((2, BN, 8), lambda i: (0, i, 0)),
            pl.BlockSpec((1, D_HID), lambda i: (0, 0)),
            pl.BlockSpec((BN, D_HID), lambda i: (i, 0)),
            pl.BlockSpec((D_HID, out_cols), lambda i: (0, 0)),
            pl.BlockSpec((out_cols, 16), lambda i: (0, 0)),
            pl.BlockSpec((8, D_HID), lambda i: (0, 0)),
        ],
        out_specs=[
            hp_spec,
            pl.BlockSpec((BN, 16), lambda i: (i, 0)),
            pl.BlockSpec((1, 16), lambda i: (0, 0)),
            pl.BlockSpec((BN, D_HID), lambda i: (i, 0)),
        ],
        out_shape=[
            jax.ShapeDtypeStruct(hp_shape, f32),
            jax.ShapeDtypeStruct((N_PAD, 16), f32),
            jax.ShapeDtypeStruct((1, 16), f32),
            jax.ShapeDtypeStruct((N_PAD, D_HID), f32),
        ],
    )(un, den, b, res, W, U, RS)


def _tc4(un3, den3, b3):
    return pl.pallas_call(
        _tc4_body,
        grid=(GRID,),
        in_specs=[
            pl.BlockSpec((2, BN, 64), lambda i: (0, i, 0)),
            pl.BlockSpec((2, BN, 8), lambda i: (0, i, 0)),
            pl.BlockSpec((1, 64), lambda i: (0, 0)),
        ],
        out_specs=pl.BlockSpec((BN, 64), lambda i: (i, 0)),
        out_shape=jax.ShapeDtypeStruct((N_PAD, 64), f32),
    )(un3, den3, b3)


_MESH = dict(core_axis_name="c", subcore_axis_name="s", num_cores=2,
             num_subcores=16)

_SC_PARAMS = pltpu.CompilerParams(needs_layout_passes=False,
                                  use_tc_tiling_on_sc=False)

ROWS_PER_TILE = N_PAD // 16


def _leaky(v):
    return jnp.where(v < 0.0, 0.2 * v, v)


KL = 64


def _light_body(asd_hbm, mm_hbm, src_hbm, dst_hbm, zr_hbm,
                ex_hbm, den_hbm,
                idx_s2, idx_d2, asb2, adb2, exb2, mmv, dacc,
                semi0, semi1, semg0, semg1):
    c = lax.axis_index("c")
    s = lax.axis_index("s")
    lane = lax.iota(i32, 16)
    rem8 = lax.rem(lane, 8)
    row2 = lane // 8
    semi = (semi0, semi1)
    semg = (semg0, semg1)

    pltpu.sync_copy(mm_hbm, mmv)
    mt = plsc.load_gather(mmv, [rem8]) + plsc.load_gather(mmv, [rem8 + 8])
    m16 = _leaky(mt)

    pltpu.sync_copy(zr_hbm, dacc.at[pl.ds(ROWS_PER_TILE * s, ROWS_PER_TILE)])
    plsc.subcore_barrier()

    tile_base = c * (E_PAD // 2) + s * (E_PAD // 32)
    n_chunks = E_PAD // 32 // KL
    max_base = E_PAD - KL

    def cbase(ci):
        return pl.multiple_of(jnp.minimum(tile_base + ci * KL, max_base), KL)

    def start_idx(ci, b):
        base = cbase(ci)
        pltpu.async_copy(src_hbm.at[pl.ds(base, KL)], idx_s2.at[b], semi[b])
        pltpu.async_copy(dst_hbm.at[pl.ds(base, KL)], idx_d2.at[b], semi[b])

    def wait_idx(b):
        pltpu.make_async_copy(src_hbm.at[pl.ds(0, KL)], idx_s2.at[b], semi[b]).wait()
        pltpu.make_async_copy(dst_hbm.at[pl.ds(0, KL)], idx_d2.at[b], semi[b]).wait()

    def start_gather(b):
        pltpu.async_copy(asd_hbm.at[idx_s2.at[b]], asb2.at[b], semg[b])
        pltpu.async_copy(asd_hbm.at[idx_d2.at[b]], adb2.at[b], semg[b])

    def wait_gather(b):
        pltpu.make_async_copy(asd_hbm.at[idx_s2.at[b]], asb2.at[b], semg[b]).wait()
        pltpu.make_async_copy(asd_hbm.at[idx_d2.at[b]], adb2.at[b], semg[b]).wait()

    start_idx(0, 0)
    wait_idx(0)
    start_gather(0)
    start_idx(1, 1)

    def chunk(g, carry):
        for b in (0, 1):
            nb = 1 - b
            wait_idx(nb)
            start_gather(nb)
            wait_gather(b)

            def pair(j, carry2):
                rowv = 2 * j + row2
                sval = plsc.load_gather(asb2.at[b], [rowv, rem8])
                dval = plsc.load_gather(adb2.at[b], [rowv, rem8 + 8])
                ex = jnp.exp(_leaky(sval + dval) - m16)
                plsc.store_scatter(exb2.at[b], [rowv, rem8], ex)
                return carry2

            lax.fori_loop(0, KL // 2, pair, 0, unroll=4)
            base = cbase(2 * g + b)
            pltpu.sync_copy(exb2.at[b], ex_hbm.at[pl.ds(base, KL)])
            pltpu.sync_copy(exb2.at[b], dacc.at[idx_d2.at[b]], add=True)
            start_idx(2 * g + b + 2, b)
        return carry

    lax.fori_loop(0, n_chunks // 2, chunk, 0)
    wait_gather(0)
    wait_idx(1)
    plsc.subcore_barrier()
    row0 = c * N_PAD + ROWS_PER_TILE * s
    pltpu.sync_copy(dacc.at[pl.ds(ROWS_PER_TILE * s, ROWS_PER_TILE)],
                    den_hbm.at[pl.ds(row0, ROWS_PER_TILE)])


def _make_light():
    return pl.kernel(
        _light_body,
        out_type=(
            jax.ShapeDtypeStruct((E_PAD, 8), f32),
            jax.ShapeDtypeStruct((2 * N_PAD, 8), f32),
        ),
        mesh=plsc.VectorSubcoreMesh(**_MESH),
        compiler_params=_SC_PARAMS,
        scratch_types=[
            pltpu.VMEM((2, KL), i32),
            pltpu.VMEM((2, KL), i32),
            pltpu.VMEM((2, KL, 16), f32),
            pltpu.VMEM((2, KL, 16), f32),
            pltpu.VMEM((2, KL, 8), f32),
            pltpu.VMEM((16,), f32),
            pltpu.VMEM_SHARED((N_PAD, 8), f32),
            pltpu.SemaphoreType.DMA,
            pltpu.SemaphoreType.DMA,
            pltpu.SemaphoreType.DMA,
            pltpu.SemaphoreType.DMA,
        ],
    )


def _heavy12_body(hp_hbm, ex_hbm, src_hbm, dst_hbm, zr_hbm,
                  out_hbm,
                  idx_s2, adj2, idx_d2, exb2, hh2, acc,
                  semi0, semi1, semg0, semg1):
    c = lax.axis_index("c")
    s = lax.axis_index("s")
    tile_base = s * (E_PAD // 16)
    n_chunks = E_PAD // 16 // K
    max_base = E_PAD - K
    semi = (semi0, semi1)
    semg = (semg0, semg1)

    def cbase(ci):
        return pl.multiple_of(jnp.minimum(tile_base + ci * K, max_base), K)

    def start_idx(ci, b):
        base = cbase(ci)
        pltpu.async_copy(src_hbm.at[pl.ds(base, K)], idx_s2.at[b], semi[b])
        pltpu.async_copy(dst_hbm.at[pl.ds(base, K)], idx_d2.at[b], semi[b])
        pltpu.async_copy(ex_hbm.at[pl.ds(base, K)], exb2.at[b], semi[b])

    def wait_idx(b):
        pltpu.make_async_copy(src_hbm.at[pl.ds(0, K)], idx_s2.at[b], semi[b]).wait()
        pltpu.make_async_copy(dst_hbm.at[pl.ds(0, K)], idx_d2.at[b], semi[b]).wait()
        pltpu.make_async_copy(ex_hbm.at[pl.ds(0, K)], exb2.at[b], semi[b]).wait()

    def start_gather(off, b):
        for j in range(K // 16):
            sl = pl.ds(16 * j, 16)
            adj2[b, sl] = jnp.minimum(idx_s2[b, sl], N_PAD - 1) + off
        pltpu.async_copy(hp_hbm.at[adj2.at[b]], hh2.at[b], semg[b])

    def wait_gather(b):
        pltpu.make_async_copy(hp_hbm.at[adj2.at[b]], hh2.at[b], semg[b]).wait()

    for r in (0, 1):
        P = 2 * c + r
        off = P * N_PAD
        pltpu.sync_copy(zr_hbm, acc.at[pl.ds(ROWS_PER_TILE * s, ROWS_PER_TILE)])
        plsc.subcore_barrier()

        col0 = jnp.full((16,), 2 * P, dtype=i32)
        col1 = col0 + 1

        start_idx(0, 0)
        wait_idx(0)
        start_gather(off, 0)
        start_idx(1, 1)

        def chunk(g, carry):
            for b in (0, 1):
                nb = 1 - b
                wait_idx(nb)
                start_gather(off, nb)
                wait_gather(b)

                def edge(k, carry2):
                    kv = jnp.full((16,), k, dtype=i32)
                    a0 = plsc.load_gather(exb2.at[b], [kv, col0])
                    a1 = plsc.load_gather(exb2.at[b], [kv, col1])
                    for j in range(8):
                        sl = pl.ds(16 * j, 16)
                        hh2[b, k, sl] = hh2[b, k, sl] * (a0 if j < 4 else a1)
                    return carry2

                lax.fori_loop(0, K, edge, 0, unroll=4)
                pltpu.sync_copy(hh2.at[b], acc.at[idx_d2.at[b]], add=True)
                start_idx(2 * g + b + 2, b)
            return carry

        lax.fori_loop(0, n_chunks // 2, chunk, 0)
        wait_gather(0)
        wait_idx(1)
        plsc.subcore_barrier()
        pltpu.sync_copy(acc.at[pl.ds(ROWS_PER_TILE * s, ROWS_PER_TILE)],
                        out_hbm.at[pl.ds(off + ROWS_PER_TILE * s, ROWS_PER_TILE)])
        plsc.subcore_barrier()


def _make_heavy12():
    return pl.kernel(
        _heavy12_body,
        out_type=jax.ShapeDtypeStruct((4 * N_PAD, 128), f32),
        mesh=plsc.VectorSubcoreMesh(**_MESH),
        compiler_params=_SC_PARAMS,
        scratch_types=[
            pltpu.VMEM((2, K), i32),
            pltpu.VMEM((2, K), i32),
            pltpu.VMEM((2, K), i32),
            pltpu.VMEM((2, K, 8), f32),
            pltpu.VMEM((2, K, 128), f32),
            pltpu.VMEM_SHARED((N_PAD, 128), f32),
            pltpu.SemaphoreType.DMA,
            pltpu.SemaphoreType.DMA,
            pltpu.SemaphoreType.DMA,
            pltpu.SemaphoreType.DMA,
        ],
    )


def _heavy3_body(h3_hbm, ex_hbm, src_hbm, dst_hbm, zr_hbm,
                 out_hbm,
                 idx_s2, idx_d2, exb2, hh2, acc,
                 semi0, semi1, semg0, semg1):
    c = lax.axis_index("c")
    s = lax.axis_index("s")
    semi = (semi0, semi1)
    semg = (semg0, semg1)
    pltpu.sync_copy(zr_hbm, acc.at[pl.ds(ROWS_PER_TILE * s, ROWS_PER_TILE)])
    plsc.subcore_barrier()

    tile_base = c * (E_PAD // 2) + s * (E_PAD // 32)
    n_chunks = E_PAD // 32 // KL
    max_base = E_PAD - KL

    def cbase(ci):
        return pl.multiple_of(jnp.minimum(tile_base + ci * KL, max_base), KL)

    def start_idx(ci, b):
        base = cbase(ci)
        pltpu.async_copy(src_hbm.at[pl.ds(base, KL)], idx_s2.at[b], semi[b])
        pltpu.async_copy(dst_hbm.at[pl.ds(base, KL)], idx_d2.at[b], semi[b])
        pltpu.async_copy(ex_hbm.at[pl.ds(base, KL)], exb2.at[b], semi[b])

    def wait_idx(b):
        pltpu.make_async_copy(src_hbm.at[pl.ds(0, KL)], idx_s2.at[b], semi[b]).wait()
        pltpu.make_async_copy(dst_hbm.at[pl.ds(0, KL)], idx_d2.at[b], semi[b]).wait()
        pltpu.make_async_copy(ex_hbm.at[pl.ds(0, KL)], exb2.at[b], semi[b]).wait()

    def start_gather(b):
        pltpu.async_copy(h3_hbm.at[idx_s2.at[b]], hh2.at[b], semg[b])

    def wait_gather(b):
        pltpu.make_async_copy(h3_hbm.at[idx_s2.at[b]], hh2.at[b], semg[b]).wait()

    col0 = jnp.full((16,), 0, dtype=i32)

    start_idx(0, 0)
    wait_idx(0)
    start_gather(0)
    start_idx(1, 1)

    def chunk(g, carry):
        for b in (0, 1):
            nb = 1 - b
            wait_idx(nb)
            start_gather(nb)
            wait_gather(b)

            def edge(k, carry2):
                kv = jnp.full((16,), k, dtype=i32)
                a0 = plsc.load_gather(exb2.at[b], [kv, col0])
                for j in range(4):
                    sl = pl.ds(16 * j, 16)
                    hh2[b, k, sl] = hh2[b, k, sl] * a0
                return carry2

            lax.fori_loop(0, KL, edge, 0, unroll=4)
            pltpu.sync_copy(hh2.at[b], acc.at[idx_d2.at[b]], add=True)
            start_idx(2 * g + b + 2, b)
        return carry

    lax.fori_loop(0, n_chunks // 2, chunk, 0)
    wait_gather(0)
    wait_idx(1)
    plsc.subcore_barrier()
    row0 = c * N_PAD + ROWS_PER_TILE * s
    pltpu.sync_copy(acc.at[pl.ds(ROWS_PER_TILE * s, ROWS_PER_TILE)],
                    out_hbm.at[pl.ds(row0, ROWS_PER_TILE)])


def _make_heavy3():
    return pl.kernel(
        _heavy3_body,
        out_type=jax.ShapeDtypeStruct((2 * N_PAD, 64), f32),
        mesh=plsc.VectorSubcoreMesh(**_MESH),
        compiler_params=_SC_PARAMS,
        scratch_types=[
            pltpu.VMEM((2, KL), i32),
            pltpu.VMEM((2, KL), i32),
            pltpu.VMEM((2, KL, 8), f32),
            pltpu.VMEM((2, KL, 64), f32),
            pltpu.VMEM_SHARED((N_PAD, 64), f32),
            pltpu.SemaphoreType.DMA,
            pltpu.SemaphoreType.DMA,
            pltpu.SemaphoreType.DMA,
            pltpu.SemaphoreType.DMA,
        ],
    )


def _proj_matrix(a_src, a_dst):
    heads, ch = a_src.shape
    d = heads * ch
    sel = (jnp.arange(d)[:, None] // ch == jnp.arange(heads)[None, :])
    us = jnp.where(sel, a_src.reshape(d)[:, None], 0.0)
    ud = jnp.where(sel, a_dst.reshape(d)[:, None], 0.0)
    z = jnp.zeros((d, 8 - heads), dtype=f32)
    return jnp.concatenate([us, z, ud, z], axis=1).astype(f32)


@jax.jit
def kernel(x, edge_index, W1, a_src1, a_dst1, b1, W2, a_src2, a_dst2, b2,
           W3, a_src3, a_dst3, b3):
    xp = jnp.zeros((N_PAD, N_FEAT), f32).at[:N].set(x)
    loop = jnp.arange(N, dtype=i32)
    padi = jnp.full((E_PAD - E_REAL,), N, dtype=i32)
    src = jnp.concatenate([edge_index[0].astype(i32), loop, padi])
    dst = jnp.concatenate([edge_index[1].astype(i32), loop, padi])

    U1 = _proj_matrix(a_src1, a_dst1)
    U2 = _proj_matrix(a_src2, a_dst2)
    U3 = _proj_matrix(a_src3, a_dst3)
    RS = (jnp.arange(D_HID)[None, :] // 64 == jnp.arange(8)[:, None]).astype(f32)

    z8 = jnp.zeros((ROWS_PER_TILE, 8), f32)
    z64 = jnp.zeros((ROWS_PER_TILE, 64), f32)
    z128 = jnp.zeros((ROWS_PER_TILE, 128), f32)

    light = _make_light()
    heavy12 = _make_heavy12()
    heavy3 = _make_heavy3()

    hp1, asd1, mm1 = _tc1(xp, W1, U1)
    ex1, den1 = light(asd1, mm1.reshape(16), src, dst, z8)
    un1 = heavy12(hp1.reshape(4 * N_PAD, 128), ex1, src, dst, z128)

    dummy_res = jnp.zeros((N_PAD, D_HID), f32)
    hp2, asd2, mm2, h1a = _tc_mid(un1.reshape(4, N_PAD, 128),
                                  den1.reshape(2, N_PAD, 8),
                                  b1.reshape(1, D_HID), dummy_res,
                                  W2, U2, RS, 512, with_residual=False)
    ex2, den2 = light(asd2, mm2.reshape(16), src, dst, z8)
    un2 = heavy12(hp2.reshape(4 * N_PAD, 128), ex2, src, dst, z128)

    h3t, asd3, mm3, _ = _tc_mid(un2.reshape(4, N_PAD, 128),
                                den2.reshape(2, N_PAD, 8),
                                b2.reshape(1, D_HID), h1a,
                                W3, U3, RS, 64, with_residual=True)
    ex3, den3 = light(asd3, mm3.reshape(16), src, dst, z8)
    un3 = heavy3(h3t, ex3, src, dst, z64)

    out = _tc4(un3.reshape(2, N_PAD, 64), den3.reshape(2, N_PAD, 8),
               b3.reshape(1, 64))
    return out[:N]


# speedup vs baseline: 37.4545x; 1.1251x over previous
import functools

import jax
import jax.numpy as jnp
from jax import lax
from jax.experimental import pallas as pl
from jax.experimental.pallas import tpu as pltpu
from jax.experimental.pallas import tpu_sc as plsc

N = 10000
E = 320000
N_FEAT = 128
D_HID = 512
N_HEAD = 8
N_CLASS = 64

N_PAD = 10240
E_REAL = E + N
K = 128
E_PAD = 32 * K * 81
BN = 512
GRID = N_PAD // BN
NEG = -1e30

f32 = jnp.float32
i32 = jnp.int32


def _mask_pad_rows(asd, i):
    rows = i * BN + lax.broadcasted_iota(i32, (BN, 16), 0)
    return jnp.where(rows < N, asd, NEG)


def _accum_mm(mm_ref, asd, i):
    cur = jnp.max(asd, axis=0, keepdims=True)

    @pl.when(i == 0)
    def _():
        mm_ref[...] = cur

    @pl.when(i > 0)
    def _():
        mm_ref[...] = jnp.maximum(mm_ref[...], cur)


def _split_pairs(h):
    return jnp.stack([h[:, 0:128], h[:, 128:256], h[:, 256:384], h[:, 384:512]],
                     axis=0)


def _tc1_body(x_ref, w_ref, u_ref, hp_ref, asd_ref, mm_ref):
    i = pl.program_id(0)
    h = jnp.dot(x_ref[...], w_ref[...], preferred_element_type=f32)
    asd = jnp.dot(h, u_ref[...], preferred_element_type=f32)
    asd_ref[...] = _mask_pad_rows(asd, i)
    hp_ref[...] = _split_pairs(h)
    _accum_mm(mm_ref, asd_ref[...], i)


def _tc_mid_body(un_ref, den_ref, b_ref, res_ref, w_ref, u_ref, rsum_ref,
                 hp_ref, asd_ref, mm_ref, hact_ref, *, with_residual):
    i = pl.program_id(0)
    un = un_ref[...]
    hcat = jnp.concatenate([un[0], un[1], un[2], un[3]], axis=1)
    d = den_ref[0] + den_ref[1]
    dexp = jnp.dot(d, rsum_ref[...], preferred_element_type=f32)
    z = hcat / (dexp + 1e-16) + b_ref[...]
    hact = jnp.where(z > 0, z, jnp.exp(jnp.minimum(z, 0.0)) - 1.0)
    if with_residual:
        hact = hact + res_ref[...]
    hact_ref[...] = hact
    h2 = jnp.dot(hact, w_ref[...], preferred_element_type=f32)
    asd = jnp.dot(h2, u_ref[...], preferred_element_type=f32)
    asd_ref[...] = _mask_pad_rows(asd, i)
    hp_ref[...] = _split_pairs(h2) if h2.shape[1] == 512 else h2
    _accum_mm(mm_ref, asd_ref[...], i)


def _tc4_body(un_ref, den_ref, b_ref, out_ref):
    s = un_ref[0] + un_ref[1]
    d = den_ref[0][:, 0:1] + den_ref[1][:, 0:1]
    out_ref[...] = s / (d + 1e-16) + b_ref[...]


def _tc1(xp, W1, U1):
    return pl.pallas_call(
        _tc1_body,
        grid=(GRID,),
        in_specs=[
            pl.BlockSpec((BN, N_FEAT), lambda i: (i, 0)),
            pl.BlockSpec((N_FEAT, D_HID), lambda i: (0, 0)),
            pl.BlockSpec((D_HID, 16), lambda i: (0, 0)),
        ],
        out_specs=[
            pl.BlockSpec((4, BN, 128), lambda i: (0, i, 0)),
            pl.BlockSpec((BN, 16), lambda i: (i, 0)),
            pl.BlockSpec((1, 16), lambda i: (0, 0)),
        ],
        out_shape=[
            jax.ShapeDtypeStruct((4, N_PAD, 128), f32),
            jax.ShapeDtypeStruct((N_PAD, 16), f32),
            jax.ShapeDtypeStruct((1, 16), f32),
        ],
    )(xp, W1, U1)


def _tc_mid(un, den, b, res, W, U, RS, out_cols, with_residual):
    n_pairs = out_cols // 128 if out_cols >= 128 else 1
    hp_shape = (4, N_PAD, 128) if out_cols == 512 else (N_PAD, out_cols)
    hp_spec = (pl.BlockSpec((4, BN, 128), lambda i: (0, i, 0))
               if out_cols == 512 else pl.BlockSpec((BN, out_cols), lambda i: (i, 0)))
    body = functools.partial(_tc_mid_body, with_residual=with_residual)
    return pl.pallas_call(
        body,
        grid=(GRID,),
        in_specs=[
            pl.BlockSpec((4, BN, 128), lambda i: (0, i, 0)),
            pl.BlockSpec((2, BN, 8), lambda i: (0, i, 0)),
            pl.BlockSpec((1, D_HID), lambda i: (0, 0)),
            pl.BlockSpec((BN, D_HID), lambda i: (i, 0)),
            pl.BlockSpec((D_HID, out_cols), lambda i: (0, 0)),
            pl.BlockSpec((out_cols, 16), lambda i: (0, 0)),
            pl.BlockSpec((8, D_HID), lambda i: (0, 0)),
        ],
        out_specs=[
            hp_spec,
            pl.BlockSpec((BN, 16), lambda i: (i, 0)),
            pl.BlockSpec((1, 16), lambda i: (0, 0)),
            pl.BlockSpec((BN, D_HID), lambda i: (i, 0)),
        ],
        out_shape=[
            jax.ShapeDtypeStruct(hp_shape, f32),
            jax.ShapeDtypeStruct((N_PAD, 16), f32),
            jax.ShapeDtypeStruct((1, 16), f32),
            jax.ShapeDtypeStruct((N_PAD, D_HID), f32),
        ],
    )(un, den, b, res, W, U, RS)


def _tc4(un3, den3, b3):
    return pl.pallas_call(
        _tc4_body,
        grid=(GRID,),
        in_specs=[
            pl.BlockSpec((2, BN, 64), lambda i: (0, i, 0)),
            pl.BlockSpec((2, BN, 8), lambda i: (0, i, 0)),
            pl.BlockSpec((1, 64), lambda i: (0, 0)),
        ],
        out_specs=pl.BlockSpec((BN, 64), lambda i: (i, 0)),
        out_shape=jax.ShapeDtypeStruct((N_PAD, 64), f32),
    )(un3, den3, b3)


_MESH = dict(core_axis_name="c", subcore_axis_name="s", num_cores=2,
             num_subcores=16)

_SC_PARAMS = pltpu.CompilerParams(needs_layout_passes=False,
                                  use_tc_tiling_on_sc=False)

ROWS_PER_TILE = N_PAD // 16


def _leaky(v):
    return jnp.where(v < 0.0, 0.2 * v, v)


KL = 64


def _light_body(asd_hbm, mm_hbm, src_hbm, dst_hbm, zr_hbm,
                ex_hbm, den_hbm,
                idx_s2, idx_d2, asb2, adb2, exb2, mmv, dacc,
                semi0, semi1, semg0, semg1):
    c = lax.axis_index("c")
    s = lax.axis_index("s")
    lane = lax.iota(i32, 16)
    rem8 = lax.rem(lane, 8)
    row2 = lane // 8
    semi = (semi0, semi1)
    semg = (semg0, semg1)

    pltpu.sync_copy(mm_hbm, mmv)
    mt = plsc.load_gather(mmv, [rem8]) + plsc.load_gather(mmv, [rem8 + 8])
    m16 = _leaky(mt)

    pltpu.sync_copy(zr_hbm, dacc.at[pl.ds(ROWS_PER_TILE * s, ROWS_PER_TILE)])
    plsc.subcore_barrier()

    tile_base = c * (E_PAD // 2) + s * (E_PAD // 32)
    n_chunks = E_PAD // 32 // KL
    max_base = E_PAD - KL

    def cbase(ci):
        return pl.multiple_of(jnp.minimum(tile_base + ci * KL, max_base), KL)

    def start_idx(ci, b):
        base = cbase(ci)
        pltpu.async_copy(src_hbm.at[pl.ds(base, KL)], idx_s2.at[b], semi[b])
        pltpu.async_copy(dst_hbm.at[pl.ds(base, KL)], idx_d2.at[b], semi[b])

    def wait_idx(b):
        pltpu.make_async_copy(src_hbm.at[pl.ds(0, KL)], idx_s2.at[b], semi[b]).wait()
        pltpu.make_async_copy(dst_hbm.at[pl.ds(0, KL)], idx_d2.at[b], semi[b]).wait()

    def start_gather(b):
        pltpu.async_copy(asd_hbm.at[idx_s2.at[b]], asb2.at[b], semg[b])
        pltpu.async_copy(asd_hbm.at[idx_d2.at[b]], adb2.at[b], semg[b])

    def wait_gather(b):
        pltpu.make_async_copy(asd_hbm.at[idx_s2.at[b]], asb2.at[b], semg[b]).wait()
        pltpu.make_async_copy(asd_hbm.at[idx_d2.at[b]], adb2.at[b], semg[b]).wait()

    start_idx(0, 0)
    wait_idx(0)
    start_gather(0)
    start_idx(1, 1)

    def chunk(g, carry):
        for b in (0, 1):
            nb = 1 - b
            wait_idx(nb)
            start_gather(nb)
            wait_gather(b)

            def pair(j, carry2):
                rowv = 2 * j + row2
                sval = plsc.load_gather(asb2.at[b], [rowv, rem8])
                dval = plsc.load_gather(adb2.at[b], [rowv, rem8 + 8])
                ex = jnp.exp(_leaky(sval + dval) - m16)
                plsc.store_scatter(exb2.at[b], [rowv, rem8], ex)
                return carry2

            lax.fori_loop(0, KL // 2, pair, 0, unroll=4)
            base = cbase(2 * g + b)
            pltpu.sync_copy(exb2.at[b], ex_hbm.at[pl.ds(base, KL)])
            pltpu.sync_copy(exb2.at[b], dacc.at[idx_d2.at[b]], add=True)
            start_idx(2 * g + b + 2, b)
        return carry

    lax.fori_loop(0, n_chunks // 2, chunk, 0)
    wait_gather(0)
    wait_idx(1)
    plsc.subcore_barrier()
    row0 = c * N_PAD + ROWS_PER_TILE * s
    pltpu.sync_copy(dacc.at[pl.ds(ROWS_PER_TILE * s, ROWS_PER_TILE)],
                    den_hbm.at[pl.ds(row0, ROWS_PER_TILE)])


def _make_light():
    return pl.kernel(
        _light_body,
        out_type=(
            jax.ShapeDtypeStruct((E_PAD, 8), f32),
            jax.ShapeDtypeStruct((2 * N_PAD, 8), f32),
        ),
        mesh=plsc.VectorSubcoreMesh(**_MESH),
        compiler_params=_SC_PARAMS,
        scratch_types=[
            pltpu.VMEM((2, KL), i32),
            pltpu.VMEM((2, KL), i32),
            pltpu.VMEM((2, KL, 16), f32),
            pltpu.VMEM((2, KL, 16), f32),
            pltpu.VMEM((2, KL, 8), f32),
            pltpu.VMEM((16,), f32),
            pltpu.VMEM_SHARED((N_PAD, 8), f32),
            pltpu.SemaphoreType.DMA,
            pltpu.SemaphoreType.DMA,
            pltpu.SemaphoreType.DMA,
            pltpu.SemaphoreType.DMA,
        ],
    )


def _heavy12_body(hp_hbm, ex_hbm, src_hbm, dst_hbm, zr_hbm,
                  out_hbm,
                  idx_s2, adj2, idx_d2, idxd_sc, exb2, hh2, acc,
                  semi0, semi1, semg0, semg1, sems0, sems1):
    c = lax.axis_index("c")
    s = lax.axis_index("s")
    tile_base = s * (E_PAD // 16)
    n_chunks = E_PAD // 16 // K
    max_base = E_PAD - K
    semi = (semi0, semi1)
    semg = (semg0, semg1)
    sems = (sems0, sems1)

    def cbase(ci):
        return pl.multiple_of(jnp.minimum(tile_base + ci * K, max_base), K)

    def start_idx(ci, b):
        base = cbase(ci)
        pltpu.async_copy(src_hbm.at[pl.ds(base, K)], idx_s2.at[b], semi[b])
        pltpu.async_copy(dst_hbm.at[pl.ds(base, K)], idx_d2.at[b], semi[b])
        pltpu.async_copy(ex_hbm.at[pl.ds(base, K)], exb2.at[b], semi[b])

    def wait_idx(b):
        pltpu.make_async_copy(src_hbm.at[pl.ds(0, K)], idx_s2.at[b], semi[b]).wait()
        pltpu.make_async_copy(dst_hbm.at[pl.ds(0, K)], idx_d2.at[b], semi[b]).wait()
        pltpu.make_async_copy(ex_hbm.at[pl.ds(0, K)], exb2.at[b], semi[b]).wait()

    def start_gather(off, b):
        for j in range(K // 16):
            sl = pl.ds(16 * j, 16)
            adj2[b, sl] = jnp.minimum(idx_s2[b, sl], N_PAD - 1) + off
        pltpu.async_copy(hp_hbm.at[adj2.at[b]], hh2.at[b], semg[b])

    def wait_gather(b):
        pltpu.make_async_copy(hp_hbm.at[adj2.at[b]], hh2.at[b], semg[b]).wait()

    def start_scatter(b):
        for j in range(K // 16):
            sl = pl.ds(16 * j, 16)
            idxd_sc[b, sl] = idx_d2[b, sl]
        pltpu.async_copy(hh2.at[b], acc.at[idxd_sc.at[b]], sems[b], add=True)

    def wait_scatter(b):
        pltpu.make_async_copy(hh2.at[b], acc.at[idxd_sc.at[b]], sems[b]).wait()

    for r in (0, 1):
        P = 2 * c + r
        off = P * N_PAD
        pltpu.sync_copy(zr_hbm, acc.at[pl.ds(ROWS_PER_TILE * s, ROWS_PER_TILE)])
        plsc.subcore_barrier()

        col0 = jnp.full((16,), 2 * P, dtype=i32)
        col1 = col0 + 1

        start_idx(0, 0)
        wait_idx(0)
        start_gather(off, 0)
        start_idx(1, 1)

        def chunk(g, carry):
            for b in (0, 1):
                nb = 1 - b
                wait_idx(nb)
                if b == 0:
                    @pl.when(g > 0)
                    def _():
                        wait_scatter(nb)
                else:
                    wait_scatter(nb)
                start_gather(off, nb)
                wait_gather(b)

                def edge(k, carry2):
                    kv = jnp.full((16,), k, dtype=i32)
                    a0 = plsc.load_gather(exb2.at[b], [kv, col0])
                    a1 = plsc.load_gather(exb2.at[b], [kv, col1])
                    for j in range(8):
                        sl = pl.ds(16 * j, 16)
                        hh2[b, k, sl] = hh2[b, k, sl] * (a0 if j < 4 else a1)
                    return carry2

                lax.fori_loop(0, K, edge, 0, unroll=4)
                start_scatter(b)
                start_idx(2 * g + b + 2, b)
            return carry

        lax.fori_loop(0, n_chunks // 2, chunk, 0)
        wait_scatter(1)
        wait_gather(0)
        wait_idx(1)
        plsc.subcore_barrier()
        pltpu.sync_copy(acc.at[pl.ds(ROWS_PER_TILE * s, ROWS_PER_TILE)],
                        out_hbm.at[pl.ds(off + ROWS_PER_TILE * s, ROWS_PER_TILE)])
        plsc.subcore_barrier()


def _make_heavy12():
    return pl.kernel(
        _heavy12_body,
        out_type=jax.ShapeDtypeStruct((4 * N_PAD, 128), f32),
        mesh=plsc.VectorSubcoreMesh(**_MESH),
        compiler_params=_SC_PARAMS,
        scratch_types=[
            pltpu.VMEM((2, K), i32),
            pltpu.VMEM((2, K), i32),
            pltpu.VMEM((2, K), i32),
            pltpu.VMEM((2, K), i32),
            pltpu.VMEM((2, K, 8), f32),
            pltpu.VMEM((2, K, 128), f32),
            pltpu.VMEM_SHARED((N_PAD, 128), f32),
            pltpu.SemaphoreType.DMA,
            pltpu.SemaphoreType.DMA,
            pltpu.SemaphoreType.DMA,
            pltpu.SemaphoreType.DMA,
            pltpu.SemaphoreType.DMA,
            pltpu.SemaphoreType.DMA,
        ],
    )


def _heavy3_body(h3_hbm, ex_hbm, src_hbm, dst_hbm, zr_hbm,
                 out_hbm,
                 idx_s2, idx_d2, exb2, hh2, acc,
                 semi0, semi1, semg0, semg1):
    c = lax.axis_index("c")
    s = lax.axis_index("s")
    semi = (semi0, semi1)
    semg = (semg0, semg1)
    pltpu.sync_copy(zr_hbm, acc.at[pl.ds(ROWS_PER_TILE * s, ROWS_PER_TILE)])
    plsc.subcore_barrier()

    tile_base = c * (E_PAD // 2) + s * (E_PAD // 32)
    n_chunks = E_PAD // 32 // KL
    max_base = E_PAD - KL

    def cbase(ci):
        return pl.multiple_of(jnp.minimum(tile_base + ci * KL, max_base), KL)

    def start_idx(ci, b):
        base = cbase(ci)
        pltpu.async_copy(src_hbm.at[pl.ds(base, KL)], idx_s2.at[b], semi[b])
        pltpu.async_copy(dst_hbm.at[pl.ds(base, KL)], idx_d2.at[b], semi[b])
        pltpu.async_copy(ex_hbm.at[pl.ds(base, KL)], exb2.at[b], semi[b])

    def wait_idx(b):
        pltpu.make_async_copy(src_hbm.at[pl.ds(0, KL)], idx_s2.at[b], semi[b]).wait()
        pltpu.make_async_copy(dst_hbm.at[pl.ds(0, KL)], idx_d2.at[b], semi[b]).wait()
        pltpu.make_async_copy(ex_hbm.at[pl.ds(0, KL)], exb2.at[b], semi[b]).wait()

    def start_gather(b):
        pltpu.async_copy(h3_hbm.at[idx_s2.at[b]], hh2.at[b], semg[b])

    def wait_gather(b):
        pltpu.make_async_copy(h3_hbm.at[idx_s2.at[b]], hh2.at[b], semg[b]).wait()

    col0 = jnp.full((16,), 0, dtype=i32)

    start_idx(0, 0)
    wait_idx(0)
    start_gather(0)
    start_idx(1, 1)

    def chunk(g, carry):
        for b in (0, 1):
            nb = 1 - b
            wait_idx(nb)
            start_gather(nb)
            wait_gather(b)

            def edge(k, carry2):
                kv = jnp.full((16,), k, dtype=i32)
                a0 = plsc.load_gather(exb2.at[b], [kv, col0])
                for j in range(4):
                    sl = pl.ds(16 * j, 16)
                    hh2[b, k, sl] = hh2[b, k, sl] * a0
                return carry2

            lax.fori_loop(0, KL, edge, 0, unroll=4)
            pltpu.sync_copy(hh2.at[b], acc.at[idx_d2.at[b]], add=True)
            start_idx(2 * g + b + 2, b)
        return carry

    lax.fori_loop(0, n_chunks // 2, chunk, 0)
    wait_gather(0)
    wait_idx(1)
    plsc.subcore_barrier()
    row0 = c * N_PAD + ROWS_PER_TILE * s
    pltpu.sync_copy(acc.at[pl.ds(ROWS_PER_TILE * s, ROWS_PER_TILE)],
                    out_hbm.at[pl.ds(row0, ROWS_PER_TILE)])


def _make_heavy3():
    return pl.kernel(
        _heavy3_body,
        out_type=jax.ShapeDtypeStruct((2 * N_PAD, 64), f32),
        mesh=plsc.VectorSubcoreMesh(**_MESH),
        compiler_params=_SC_PARAMS,
        scratch_types=[
            pltpu.VMEM((2, KL), i32),
            pltpu.VMEM((2, KL), i32),
            pltpu.VMEM((2, KL, 8), f32),
            pltpu.VMEM((2, KL, 64), f32),
            pltpu.VMEM_SHARED((N_PAD, 64), f32),
            pltpu.SemaphoreType.DMA,
            pltpu.SemaphoreType.DMA,
            pltpu.SemaphoreType.DMA,
            pltpu.SemaphoreType.DMA,
        ],
    )


def _proj_matrix(a_src, a_dst):
    heads, ch = a_src.shape
    d = heads * ch
    sel = (jnp.arange(d)[:, None] // ch == jnp.arange(heads)[None, :])
    us = jnp.where(sel, a_src.reshape(d)[:, None], 0.0)
    ud = jnp.where(sel, a_dst.reshape(d)[:, None], 0.0)
    z = jnp.zeros((d, 8 - heads), dtype=f32)
    return jnp.concatenate([us, z, ud, z], axis=1).astype(f32)


@jax.jit
def kernel(x, edge_index, W1, a_src1, a_dst1, b1, W2, a_src2, a_dst2, b2,
           W3, a_src3, a_dst3, b3):
    xp = jnp.zeros((N_PAD, N_FEAT), f32).at[:N].set(x)
    loop = jnp.arange(N, dtype=i32)
    padi = jnp.full((E_PAD - E_REAL,), N, dtype=i32)
    src = jnp.concatenate([edge_index[0].astype(i32), loop, padi])
    dst = jnp.concatenate([edge_index[1].astype(i32), loop, padi])

    U1 = _proj_matrix(a_src1, a_dst1)
    U2 = _proj_matrix(a_src2, a_dst2)
    U3 = _proj_matrix(a_src3, a_dst3)
    RS = (jnp.arange(D_HID)[None, :] // 64 == jnp.arange(8)[:, None]).astype(f32)

    z8 = jnp.zeros((ROWS_PER_TILE, 8), f32)
    z64 = jnp.zeros((ROWS_PER_TILE, 64), f32)
    z128 = jnp.zeros((ROWS_PER_TILE, 128), f32)

    light = _make_light()
    heavy12 = _make_heavy12()
    heavy3 = _make_heavy3()

    hp1, asd1, mm1 = _tc1(xp, W1, U1)
    ex1, den1 = light(asd1, mm1.reshape(16), src, dst, z8)
    un1 = heavy12(hp1.reshape(4 * N_PAD, 128), ex1, src, dst, z128)

    dummy_res = jnp.zeros((N_PAD, D_HID), f32)
    hp2, asd2, mm2, h1a = _tc_mid(un1.reshape(4, N_PAD, 128),
                                  den1.reshape(2, N_PAD, 8),
                                  b1.reshape(1, D_HID), dummy_res,
                                  W2, U2, RS, 512, with_residual=False)
    ex2, den2 = light(asd2, mm2.reshape(16), src, dst, z8)
    un2 = heavy12(hp2.reshape(4 * N_PAD, 128), ex2, src, dst, z128)

    h3t, asd3, mm3, _ = _tc_mid(un2.reshape(4, N_PAD, 128),
                                den2.reshape(2, N_PAD, 8),
                                b2.reshape(1, D_HID), h1a,
                                W3, U3, RS, 64, with_residual=True)
    ex3, den3 = light(asd3, mm3.reshape(16), src, dst, z8)
    un3 = heavy3(h3t, ex3, src, dst, z64)

    out = _tc4(un3.reshape(2, N_PAD, 64), den3.reshape(2, N_PAD, 8),
               b3.reshape(1, 64))
    return out[:N]


# speedup vs baseline: 37.8560x; 1.0107x over previous
import functools

import jax
import jax.numpy as jnp
from jax import lax
from jax.experimental import pallas as pl
from jax.experimental.pallas import tpu as pltpu
from jax.experimental.pallas import tpu_sc as plsc

N = 10000
E = 320000
N_FEAT = 128
D_HID = 512
N_HEAD = 8
N_CLASS = 64

N_PAD = 10240
E_REAL = E + N
K = 128
E_PAD = 32 * K * 81
BN = 512
GRID = N_PAD // BN
NEG = -1e30

f32 = jnp.float32
i32 = jnp.int32


def _mask_pad_rows(asd, i):
    rows = i * BN + lax.broadcasted_iota(i32, (BN, 16), 0)
    return jnp.where(rows < N, asd, NEG)


def _accum_mm(mm_ref, asd, i):
    cur = jnp.max(asd, axis=0, keepdims=True)

    @pl.when(i == 0)
    def _():
        mm_ref[...] = cur

    @pl.when(i > 0)
    def _():
        mm_ref[...] = jnp.maximum(mm_ref[...], cur)


def _split_pairs(h):
    return jnp.stack([h[:, 0:128], h[:, 128:256], h[:, 256:384], h[:, 384:512]],
                     axis=0)


def _tc1_body(x_ref, w_ref, u_ref, hp_ref, asd_ref, mm_ref):
    i = pl.program_id(0)
    h = jnp.dot(x_ref[...], w_ref[...], preferred_element_type=f32)
    asd = jnp.dot(h, u_ref[...], preferred_element_type=f32)
    asd_ref[...] = _mask_pad_rows(asd, i)
    hp_ref[...] = _split_pairs(h)
    _accum_mm(mm_ref, asd_ref[...], i)


def _tc_mid_body(un_ref, den_ref, b_ref, res_ref, w_ref, u_ref, rsum_ref,
                 hp_ref, asd_ref, mm_ref, hact_ref, *, with_residual):
    i = pl.program_id(0)
    un = un_ref[...]
    hcat = jnp.concatenate([un[0], un[1], un[2], un[3]], axis=1)
    d = den_ref[0] + den_ref[1]
    dexp = jnp.dot(d, rsum_ref[...], preferred_element_type=f32)
    z = hcat / (dexp + 1e-16) + b_ref[...]
    hact = jnp.where(z > 0, z, jnp.exp(jnp.minimum(z, 0.0)) - 1.0)
    if with_residual:
        hact = hact + res_ref[...]
    hact_ref[...] = hact
    h2 = jnp.dot(hact, w_ref[...], preferred_element_type=f32)
    asd = jnp.dot(h2, u_ref[...], preferred_element_type=f32)
    asd_ref[...] = _mask_pad_rows(asd, i)
    hp_ref[...] = _split_pairs(h2) if h2.shape[1] == 512 else h2
    _accum_mm(mm_ref, asd_ref[...], i)


def _tc4_body(un_ref, den_ref, b_ref, out_ref):
    s = un_ref[0] + un_ref[1]
    d = den_ref[0][:, 0:1] + den_ref[1][:, 0:1]
    out_ref[...] = s / (d + 1e-16) + b_ref[...]


def _tc1(xp, W1, U1):
    return pl.pallas_call(
        _tc1_body,
        grid=(GRID,),
        in_specs=[
            pl.BlockSpec((BN, N_FEAT), lambda i: (i, 0)),
            pl.BlockSpec((N_FEAT, D_HID), lambda i: (0, 0)),
            pl.BlockSpec((D_HID, 16), lambda i: (0, 0)),
        ],
        out_specs=[
            pl.BlockSpec((4, BN, 128), lambda i: (0, i, 0)),
            pl.BlockSpec((BN, 16), lambda i: (i, 0)),
            pl.BlockSpec((1, 16), lambda i: (0, 0)),
        ],
        out_shape=[
            jax.ShapeDtypeStruct((4, N_PAD, 128), f32),
            jax.ShapeDtypeStruct((N_PAD, 16), f32),
            jax.ShapeDtypeStruct((1, 16), f32),
        ],
    )(xp, W1, U1)


def _tc_mid(un, den, b, res, W, U, RS, out_cols, with_residual):
    n_pairs = out_cols // 128 if out_cols >= 128 else 1
    hp_shape = (4, N_PAD, 128) if out_cols == 512 else (N_PAD, out_cols)
    hp_spec = (pl.BlockSpec((4, BN, 128), lambda i: (0, i, 0))
               if out_cols == 512 else pl.BlockSpec((BN, out_cols), lambda i: (i, 0)))
    body = functools.partial(_tc_mid_body, with_residual=with_residual)
    return pl.pallas_call(
        body,
        grid=(GRID,),
        in_specs=[
            pl.BlockSpec((4, BN, 128), lambda i: (0, i, 0)),
            pl.BlockSpec((2, BN, 8), lambda i: (0, i, 0)),
            pl.BlockSpec((1, D_HID), lambda i: (0, 0)),
            pl.BlockSpec((BN, D_HID), lambda i: (i, 0)),
            pl.BlockSpec((D_HID, out_cols), lambda i: (0, 0)),
            pl.BlockSpec((out_cols, 16), lambda i: (0, 0)),
            pl.BlockSpec((8, D_HID), lambda i: (0, 0)),
        ],
        out_specs=[
            hp_spec,
            pl.BlockSpec((BN, 16), lambda i: (i, 0)),
            pl.BlockSpec((1, 16), lambda i: (0, 0)),
            pl.BlockSpec((BN, D_HID), lambda i: (i, 0)),
        ],
        out_shape=[
            jax.ShapeDtypeStruct(hp_shape, f32),
            jax.ShapeDtypeStruct((N_PAD, 16), f32),
            jax.ShapeDtypeStruct((1, 16), f32),
            jax.ShapeDtypeStruct((N_PAD, D_HID), f32),
        ],
    )(un, den, b, res, W, U, RS)


def _tc4(un3, den3, b3):
    return pl.pallas_call(
        _tc4_body,
        grid=(GRID,),
        in_specs=[
            pl.BlockSpec((2, BN, 64), lambda i: (0, i, 0)),
            pl.BlockSpec((2, BN, 8), lambda i: (0, i, 0)),
            pl.BlockSpec((1, 64), lambda i: (0, 0)),
        ],
        out_specs=pl.BlockSpec((BN, 64), lambda i: (i, 0)),
        out_shape=jax.ShapeDtypeStruct((N_PAD, 64), f32),
    )(un3, den3, b3)


_MESH = dict(core_axis_name="c", subcore_axis_name="s", num_cores=2,
             num_subcores=16)

_SC_PARAMS = pltpu.CompilerParams(needs_layout_passes=False,
                                  use_tc_tiling_on_sc=False)

ROWS_PER_TILE = N_PAD // 16


def _leaky(v):
    return jnp.where(v < 0.0, 0.2 * v, v)


KL = 64


def _light_body(asd_hbm, mm_hbm, src_hbm, dst_hbm, zr_hbm,
                ex_hbm, den_hbm,
                idx_s2, idx_d2, idxd_sc, asb2, adb2, exb2, mmv, dacc,
                semi0, semi1, semg0, semg1, semo0, semo1):
    c = lax.axis_index("c")
    s = lax.axis_index("s")
    lane = lax.iota(i32, 16)
    rem8 = lax.rem(lane, 8)
    row2 = lane // 8
    semi = (semi0, semi1)
    semg = (semg0, semg1)
    semo = (semo0, semo1)

    pltpu.sync_copy(mm_hbm, mmv)
    mt = plsc.load_gather(mmv, [rem8]) + plsc.load_gather(mmv, [rem8 + 8])
    m16 = _leaky(mt)

    pltpu.sync_copy(zr_hbm, dacc.at[pl.ds(ROWS_PER_TILE * s, ROWS_PER_TILE)])
    plsc.subcore_barrier()

    tile_base = c * (E_PAD // 2) + s * (E_PAD // 32)
    n_chunks = E_PAD // 32 // KL
    max_base = E_PAD - KL

    def cbase(ci):
        return pl.multiple_of(jnp.minimum(tile_base + ci * KL, max_base), KL)

    def start_idx(ci, b):
        base = cbase(ci)
        pltpu.async_copy(src_hbm.at[pl.ds(base, KL)], idx_s2.at[b], semi[b])
        pltpu.async_copy(dst_hbm.at[pl.ds(base, KL)], idx_d2.at[b], semi[b])

    def wait_idx(b):
        pltpu.make_async_copy(src_hbm.at[pl.ds(0, KL)], idx_s2.at[b], semi[b]).wait()
        pltpu.make_async_copy(dst_hbm.at[pl.ds(0, KL)], idx_d2.at[b], semi[b]).wait()

    def start_gather(b):
        pltpu.async_copy(asd_hbm.at[idx_s2.at[b]], asb2.at[b], semg[b])
        pltpu.async_copy(asd_hbm.at[idx_d2.at[b]], adb2.at[b], semg[b])

    def wait_gather(b):
        pltpu.make_async_copy(asd_hbm.at[idx_s2.at[b]], asb2.at[b], semg[b]).wait()
        pltpu.make_async_copy(asd_hbm.at[idx_d2.at[b]], adb2.at[b], semg[b]).wait()

    def wait_out(b):
        pltpu.make_async_copy(exb2.at[b], ex_hbm.at[pl.ds(0, KL)], semo[b]).wait()
        pltpu.make_async_copy(exb2.at[b], dacc.at[idxd_sc.at[b]], semo[b]).wait()

    start_idx(0, 0)
    wait_idx(0)
    start_gather(0)
    start_idx(1, 1)

    def chunk(g, carry):
        for b in (0, 1):
            nb = 1 - b
            wait_idx(nb)
            start_gather(nb)
            wait_gather(b)

            def pair(j, carry2):
                rowv = 2 * j + row2
                sval = plsc.load_gather(asb2.at[b], [rowv, rem8])
                dval = plsc.load_gather(adb2.at[b], [rowv, rem8 + 8])
                ex = jnp.exp(_leaky(sval + dval) - m16)
                plsc.store_scatter(exb2.at[b], [rowv, rem8], ex)
                return carry2

            lax.fori_loop(0, KL // 2, pair, 0, unroll=4)
            base = cbase(2 * g + b)
            pltpu.sync_copy(exb2.at[b], ex_hbm.at[pl.ds(base, KL)])
            pltpu.sync_copy(exb2.at[b], dacc.at[idx_d2.at[b]], add=True)
            start_idx(2 * g + b + 2, b)
        return carry

    lax.fori_loop(0, n_chunks // 2, chunk, 0)
    wait_gather(0)
    wait_idx(1)
    plsc.subcore_barrier()
    row0 = c * N_PAD + ROWS_PER_TILE * s
    pltpu.sync_copy(dacc.at[pl.ds(ROWS_PER_TILE * s, ROWS_PER_TILE)],
                    den_hbm.at[pl.ds(row0, ROWS_PER_TILE)])


def _make_light():
    return pl.kernel(
        _light_body,
        out_type=(
            jax.ShapeDtypeStruct((E_PAD, 8), f32),
            jax.ShapeDtypeStruct((2 * N_PAD, 8), f32),
        ),
        mesh=plsc.VectorSubcoreMesh(**_MESH),
        compiler_params=_SC_PARAMS,
        scratch_types=[
            pltpu.VMEM((2, KL), i32),
            pltpu.VMEM((2, KL), i32),
            pltpu.VMEM((2, KL), i32),
            pltpu.VMEM((2, KL, 16), f32),
            pltpu.VMEM((2, KL, 16), f32),
            pltpu.VMEM((2, KL, 8), f32),
            pltpu.VMEM((16,), f32),
            pltpu.VMEM_SHARED((N_PAD, 8), f32),
            pltpu.SemaphoreType.DMA,
            pltpu.SemaphoreType.DMA,
            pltpu.SemaphoreType.DMA,
            pltpu.SemaphoreType.DMA,
            pltpu.SemaphoreType.DMA,
            pltpu.SemaphoreType.DMA,
        ],
    )


def _heavy12_body(hp_hbm, ex_hbm, src_hbm, dst_hbm, zr_hbm,
                  out_hbm,
                  idx_s2, adj2, idx_d2, idxd_sc, exb2, hh2, acc,
                  semi0, semi1, semg0, semg1, sems0, sems1):
    c = lax.axis_index("c")
    s = lax.axis_index("s")
    tile_base = s * (E_PAD // 16)
    n_chunks = E_PAD // 16 // K
    max_base = E_PAD - K
    semi = (semi0, semi1)
    semg = (semg0, semg1)
    sems = (sems0, sems1)

    def cbase(ci):
        return pl.multiple_of(jnp.minimum(tile_base + ci * K, max_base), K)

    def start_idx(ci, b):
        base = cbase(ci)
        pltpu.async_copy(src_hbm.at[pl.ds(base, K)], idx_s2.at[b], semi[b])
        pltpu.async_copy(dst_hbm.at[pl.ds(base, K)], idx_d2.at[b], semi[b])
        pltpu.async_copy(ex_hbm.at[pl.ds(base, K)], exb2.at[b], semi[b])

    def wait_idx(b):
        pltpu.make_async_copy(src_hbm.at[pl.ds(0, K)], idx_s2.at[b], semi[b]).wait()
        pltpu.make_async_copy(dst_hbm.at[pl.ds(0, K)], idx_d2.at[b], semi[b]).wait()
        pltpu.make_async_copy(ex_hbm.at[pl.ds(0, K)], exb2.at[b], semi[b]).wait()

    def start_gather(off, b):
        for j in range(K // 16):
            sl = pl.ds(16 * j, 16)
            adj2[b, sl] = jnp.minimum(idx_s2[b, sl], N_PAD - 1) + off
        pltpu.async_copy(hp_hbm.at[adj2.at[b]], hh2.at[b], semg[b])

    def wait_gather(b):
        pltpu.make_async_copy(hp_hbm.at[adj2.at[b]], hh2.at[b], semg[b]).wait()

    def start_scatter(b):
        for j in range(K // 16):
            sl = pl.ds(16 * j, 16)
            idxd_sc[b, sl] = idx_d2[b, sl]
        pltpu.async_copy(hh2.at[b], acc.at[idxd_sc.at[b]], sems[b], add=True)

    def wait_scatter(b):
        pltpu.make_async_copy(hh2.at[b], acc.at[idxd_sc.at[b]], sems[b]).wait()

    for r in (0, 1):
        P = 2 * c + r
        off = P * N_PAD
        pltpu.sync_copy(zr_hbm, acc.at[pl.ds(ROWS_PER_TILE * s, ROWS_PER_TILE)])
        plsc.subcore_barrier()

        col0 = jnp.full((16,), 2 * P, dtype=i32)
        col1 = col0 + 1

        start_idx(0, 0)
        wait_idx(0)
        start_gather(off, 0)
        start_idx(1, 1)

        def chunk(g, carry):
            for b in (0, 1):
                nb = 1 - b
                wait_idx(nb)
                if b == 0:
                    @pl.when(g > 0)
                    def _():
                        wait_scatter(nb)
                else:
                    wait_scatter(nb)
                start_gather(off, nb)
                wait_gather(b)

                def edge(k, carry2):
                    kv = jnp.full((16,), k, dtype=i32)
                    a0 = plsc.load_gather(exb2.at[b], [kv, col0])
                    a1 = plsc.load_gather(exb2.at[b], [kv, col1])
                    for j in range(8):
                        sl = pl.ds(16 * j, 16)
                        hh2[b, k, sl] = hh2[b, k, sl] * (a0 if j < 4 else a1)
                    return carry2

                lax.fori_loop(0, K, edge, 0, unroll=4)
                start_scatter(b)
                start_idx(2 * g + b + 2, b)
            return carry

        lax.fori_loop(0, n_chunks // 2, chunk, 0)
        wait_scatter(1)
        wait_gather(0)
        wait_idx(1)
        plsc.subcore_barrier()
        pltpu.sync_copy(acc.at[pl.ds(ROWS_PER_TILE * s, ROWS_PER_TILE)],
                        out_hbm.at[pl.ds(off + ROWS_PER_TILE * s, ROWS_PER_TILE)])
        plsc.subcore_barrier()


def _make_heavy12():
    return pl.kernel(
        _heavy12_body,
        out_type=jax.ShapeDtypeStruct((4 * N_PAD, 128), f32),
        mesh=plsc.VectorSubcoreMesh(**_MESH),
        compiler_params=_SC_PARAMS,
        scratch_types=[
            pltpu.VMEM((2, K), i32),
            pltpu.VMEM((2, K), i32),
            pltpu.VMEM((2, K), i32),
            pltpu.VMEM((2, K), i32),
            pltpu.VMEM((2, K, 8), f32),
            pltpu.VMEM((2, K, 128), f32),
            pltpu.VMEM_SHARED((N_PAD, 128), f32),
            pltpu.SemaphoreType.DMA,
            pltpu.SemaphoreType.DMA,
            pltpu.SemaphoreType.DMA,
            pltpu.SemaphoreType.DMA,
            pltpu.SemaphoreType.DMA,
            pltpu.SemaphoreType.DMA,
        ],
    )


def _heavy3_body(h3_hbm, ex_hbm, src_hbm, dst_hbm, zr_hbm,
                 out_hbm,
                 idx_s2, idx_d2, idxd_sc, exb2, hh2, acc,
                 semi0, semi1, semg0, semg1, sems0, sems1):
    c = lax.axis_index("c")
    s = lax.axis_index("s")
    semi = (semi0, semi1)
    semg = (semg0, semg1)
    sems = (sems0, sems1)
    pltpu.sync_copy(zr_hbm, acc.at[pl.ds(ROWS_PER_TILE * s, ROWS_PER_TILE)])
    plsc.subcore_barrier()

    tile_base = c * (E_PAD // 2) + s * (E_PAD // 32)
    n_chunks = E_PAD // 32 // KL
    max_base = E_PAD - KL

    def cbase(ci):
        return pl.multiple_of(jnp.minimum(tile_base + ci * KL, max_base), KL)

    def start_idx(ci, b):
        base = cbase(ci)
        pltpu.async_copy(src_hbm.at[pl.ds(base, KL)], idx_s2.at[b], semi[b])
        pltpu.async_copy(dst_hbm.at[pl.ds(base, KL)], idx_d2.at[b], semi[b])
        pltpu.async_copy(ex_hbm.at[pl.ds(base, KL)], exb2.at[b], semi[b])

    def wait_idx(b):
        pltpu.make_async_copy(src_hbm.at[pl.ds(0, KL)], idx_s2.at[b], semi[b]).wait()
        pltpu.make_async_copy(dst_hbm.at[pl.ds(0, KL)], idx_d2.at[b], semi[b]).wait()
        pltpu.make_async_copy(ex_hbm.at[pl.ds(0, KL)], exb2.at[b], semi[b]).wait()

    def start_gather(b):
        pltpu.async_copy(h3_hbm.at[idx_s2.at[b]], hh2.at[b], semg[b])

    def wait_gather(b):
        pltpu.make_async_copy(h3_hbm.at[idx_s2.at[b]], hh2.at[b], semg[b]).wait()

    def start_scatter(b):
        for j in range(KL // 16):
            sl = pl.ds(16 * j, 16)
            idxd_sc[b, sl] = idx_d2[b, sl]
        pltpu.async_copy(hh2.at[b], acc.at[idxd_sc.at[b]], sems[b], add=True)

    def wait_scatter(b):
        pltpu.make_async_copy(hh2.at[b], acc.at[idxd_sc.at[b]], sems[b]).wait()

    col0 = jnp.full((16,), 0, dtype=i32)

    start_idx(0, 0)
    wait_idx(0)
    start_gather(0)
    start_idx(1, 1)

    def chunk(g, carry):
        for b in (0, 1):
            nb = 1 - b
            wait_idx(nb)
            if b == 0:
                @pl.when(g > 0)
                def _():
                    wait_scatter(nb)
            else:
                wait_scatter(nb)
            start_gather(nb)
            wait_gather(b)

            def edge(k, carry2):
                kv = jnp.full((16,), k, dtype=i32)
                a0 = plsc.load_gather(exb2.at[b], [kv, col0])
                for j in range(4):
                    sl = pl.ds(16 * j, 16)
                    hh2[b, k, sl] = hh2[b, k, sl] * a0
                return carry2

            lax.fori_loop(0, KL, edge, 0, unroll=4)
            start_scatter(b)
            start_idx(2 * g + b + 2, b)
        return carry

    lax.fori_loop(0, n_chunks // 2, chunk, 0)
    wait_scatter(1)
    wait_gather(0)
    wait_idx(1)
    plsc.subcore_barrier()
    row0 = c * N_PAD + ROWS_PER_TILE * s
    pltpu.sync_copy(acc.at[pl.ds(ROWS_PER_TILE * s, ROWS_PER_TILE)],
                    out_hbm.at[pl.ds(row0, ROWS_PER_TILE)])


def _make_heavy3():
    return pl.kernel(
        _heavy3_body,
        out_type=jax.ShapeDtypeStruct((2 * N_PAD, 64), f32),
        mesh=plsc.VectorSubcoreMesh(**_MESH),
        compiler_params=_SC_PARAMS,
        scratch_types=[
            pltpu.VMEM((2, KL), i32),
            pltpu.VMEM((2, KL), i32),
            pltpu.VMEM((2, KL), i32),
            pltpu.VMEM((2, KL, 8), f32),
            pltpu.VMEM((2, KL, 64), f32),
            pltpu.VMEM_SHARED((N_PAD, 64), f32),
            pltpu.SemaphoreType.DMA,
            pltpu.SemaphoreType.DMA,
            pltpu.SemaphoreType.DMA,
            pltpu.SemaphoreType.DMA,
            pltpu.SemaphoreType.DMA,
            pltpu.SemaphoreType.DMA,
        ],
    )


def _proj_matrix(a_src, a_dst):
    heads, ch = a_src.shape
    d = heads * ch
    sel = (jnp.arange(d)[:, None] // ch == jnp.arange(heads)[None, :])
    us = jnp.where(sel, a_src.reshape(d)[:, None], 0.0)
    ud = jnp.where(sel, a_dst.reshape(d)[:, None], 0.0)
    z = jnp.zeros((d, 8 - heads), dtype=f32)
    return jnp.concatenate([us, z, ud, z], axis=1).astype(f32)


@jax.jit
def kernel(x, edge_index, W1, a_src1, a_dst1, b1, W2, a_src2, a_dst2, b2,
           W3, a_src3, a_dst3, b3):
    xp = jnp.zeros((N_PAD, N_FEAT), f32).at[:N].set(x)
    loop = jnp.arange(N, dtype=i32)
    padi = jnp.full((E_PAD - E_REAL,), N, dtype=i32)
    src = jnp.concatenate([edge_index[0].astype(i32), loop, padi])
    dst = jnp.concatenate([edge_index[1].astype(i32), loop, padi])

    U1 = _proj_matrix(a_src1, a_dst1)
    U2 = _proj_matrix(a_src2, a_dst2)
    U3 = _proj_matrix(a_src3, a_dst3)
    RS = (jnp.arange(D_HID)[None, :] // 64 == jnp.arange(8)[:, None]).astype(f32)

    z8 = jnp.zeros((ROWS_PER_TILE, 8), f32)
    z64 = jnp.zeros((ROWS_PER_TILE, 64), f32)
    z128 = jnp.zeros((ROWS_PER_TILE, 128), f32)

    light = _make_light()
    heavy12 = _make_heavy12()
    heavy3 = _make_heavy3()

    hp1, asd1, mm1 = _tc1(xp, W1, U1)
    ex1, den1 = light(asd1, mm1.reshape(16), src, dst, z8)
    un1 = heavy12(hp1.reshape(4 * N_PAD, 128), ex1, src, dst, z128)

    dummy_res = jnp.zeros((N_PAD, D_HID), f32)
    hp2, asd2, mm2, h1a = _tc_mid(un1.reshape(4, N_PAD, 128),
                                  den1.reshape(2, N_PAD, 8),
                                  b1.reshape(1, D_HID), dummy_res,
                                  W2, U2, RS, 512, with_residual=False)
    ex2, den2 = light(asd2, mm2.reshape(16), src, dst, z8)
    un2 = heavy12(hp2.reshape(4 * N_PAD, 128), ex2, src, dst, z128)

    h3t, asd3, mm3, _ = _tc_mid(un2.reshape(4, N_PAD, 128),
                                den2.reshape(2, N_PAD, 8),
                                b2.reshape(1, D_HID), h1a,
                                W3, U3, RS, 64, with_residual=True)
    ex3, den3 = light(asd3, mm3.reshape(16), src, dst, z8)
    un3 = heavy3(h3t, ex3, src, dst, z64)

    out = _tc4(un3.reshape(2, N_PAD, 64), den3.reshape(2, N_PAD, 8),
               b3.reshape(1, 64))
    return out[:N]
